# baseline (speedup 1.0000x reference)
import functools

import jax
import jax.numpy as jnp
from jax import lax
from jax.experimental import pallas as pl
from jax.experimental.pallas import tpu as pltpu
from jax.experimental.pallas import tpu_sc as plsc

N = 1024
NI = 4
NT = NI * N
KE = 27
KP = 28
KO = 24
NB = 8
BN = N // NB
NEG = -3.4e38
EPS = 1e-5


def _leaky(x):
    return jnp.where(x >= 0, x, 0.2 * x)


def _dot16(a, b):
    return lax.dot_general(a.astype(jnp.bfloat16), b.astype(jnp.bfloat16),
                           (((1,), (1,)), ((), ())),
                           preferred_element_type=jnp.float32)


def _diag_row(nn2):
    rowi = lax.broadcasted_iota(jnp.int32, (N, N), 0)
    coli = lax.broadcasted_iota(jnp.int32, (N, N), 1)
    d = jnp.where(rowi == coli, jnp.broadcast_to(nn2, (N, N)), 0.0)
    return jnp.sum(d, axis=0, keepdims=True)


RB = 8


def _topk_store(d_scr, idx_ref, k, k_pad, off):
    cols = lax.broadcasted_iota(jnp.int32, (RB, N), 1)
    tpos = lax.broadcasted_iota(jnp.int32, (RB, k_pad), 1)

    def chunk(i, carry):
        d = d_scr[pl.ds(i * RB, RB), :]
        acc = jnp.zeros((RB, k_pad), jnp.int32)
        first = None
        for t in range(k):
            m = jnp.max(d, axis=1, keepdims=True)
            cand = jnp.where(d >= m, cols, jnp.int32(2 * N))
            am = jnp.min(cand, axis=1, keepdims=True)
            acc = jnp.where(tpos == t, am, acc)
            d = jnp.where(cols == am, NEG, d)
            if t == 0:
                first = am
        if k_pad > k:
            acc = jnp.where(tpos >= k, first, acc)
        idx_ref[0, pl.ds(i * RB, RB), :] = acc + off
        return carry

    lax.fori_loop(0, N // RB, chunk, 0)


def _score_blocks(d_scr, load_row_blk, Full, nn_ref_row, nn_query_col):
    for ib in range(N // 128):
        inner = -2.0 * _dot16(load_row_blk(ib), Full)
        d_scr[pl.ds(ib * 128, 128), :] = (
            (-nn_ref_row) - inner) - nn_query_col[ib * 128:(ib + 1) * 128, :]


def _k1_body(p_ref, idx_ref, t0_ref, d_scr):
    q = pl.program_id(0)
    P = p_ref[0]
    nn2 = jnp.sum(P * P, axis=1, keepdims=True)
    nrow = _diag_row(nn2)
    _score_blocks(d_scr, lambda ib: p_ref[0, pl.ds(ib * 128, 128), :],
                  P, nrow, nn2)
    _topk_store(d_scr, idx_ref, KE, KP, q * N)
    t0_ref[0] = jnp.concatenate([P, jnp.zeros((N, 125), jnp.float32)], axis=1)


def _k1(P):
    return pl.pallas_call(
        _k1_body,
        grid=(NI,),
        in_specs=[pl.BlockSpec((1, N, 3), lambda q: (q, 0, 0))],
        out_specs=[
            pl.BlockSpec((1, N, KP), lambda q: (q, 0, 0)),
            pl.BlockSpec((1, N, 128), lambda q: (q, 0, 0)),
        ],
        out_shape=[
            jax.ShapeDtypeStruct((NI, N, KP), jnp.int32),
            jax.ShapeDtypeStruct((NI, N, 128), jnp.float32),
        ],
        scratch_shapes=[pltpu.VMEM((N, N), jnp.float32)],
    )(P)


def _sc_gather(tab, idx_flat, kp, TW):
    NW = 32
    NPW = NT // NW
    G = 4
    NG = NPW // G
    mesh = plsc.VectorSubcoreMesh(core_axis_name="c", subcore_axis_name="s")

    @functools.partial(
        pl.kernel,
        mesh=mesh,
        out_type=jax.ShapeDtypeStruct((NT * kp, TW), jnp.float32),
        scratch_types=[
            pltpu.VMEM((NPW * kp,), jnp.int32),
            pltpu.VMEM((G * kp, TW), jnp.float32),
            pltpu.VMEM((G * kp, TW), jnp.float32),
            pltpu.SemaphoreType.DMA,
            pltpu.SemaphoreType.DMA,
        ],
    )
    def sc_k(tab_hbm, idx_hbm, out_hbm, idx_v, gb0, gb1, sem0, sem1):
        wid = lax.axis_index("s") * 2 + lax.axis_index("c")
        base = wid * NPW
        pltpu.sync_copy(idx_hbm.at[pl.ds(base * kp, NPW * kp)], idx_v)
        rows = G * kp

        def fire(g, gbuf, sem):
            return pltpu.async_copy(
                tab_hbm.at[idx_v.at[pl.ds(g * rows, rows)]], gbuf, sem)

        def drain(gbuf, sem):
            pltpu.make_async_copy(tab_hbm.at[pl.ds(0, rows)], gbuf, sem).wait()

        fire(0, gb0, sem0)

        def group(h, carry):
            g = h * 2
            fire(g + 1, gb1, sem1)
            drain(gb0, sem0)
            pltpu.sync_copy(gb0, out_hbm.at[pl.ds((base + g * G) * kp, rows)])
            fire(g + 2, gb0, sem0)
            drain(gb1, sem1)
            pltpu.sync_copy(
                gb1, out_hbm.at[pl.ds((base + (g + 1) * G) * kp, rows)])
            return carry

        lax.fori_loop(0, NG // 2 - 1, group, 0)
        g = NG - 2
        fire(g + 1, gb1, sem1)
        drain(gb0, sem0)
        pltpu.sync_copy(gb0, out_hbm.at[pl.ds((base + g * G) * kp, rows)])
        drain(gb1, sem1)
        pltpu.sync_copy(gb1, out_hbm.at[pl.ds((base + (g + 1) * G) * kp, rows)])

    return sc_k(tab, idx_flat)


def _reduce_write(out2, kp, kr, O, q, ib, mx_ref, mn_ref, s_ref):
    out3 = out2.reshape(BN, kp, O)
    v0 = out3[:, 0, :]
    mx = v0
    mn = v0
    s = v0
    qq = v0 * v0
    for j in range(1, kr):
        v = out3[:, j, :]
        mx = jnp.maximum(mx, v)
        mn = jnp.minimum(mn, v)
        s = s + v
        qq = qq + v * v
    mx_ref[...] = mx
    mn_ref[...] = mn
    part = jnp.concatenate(
        [jnp.sum(s, axis=0, keepdims=True),
         jnp.sum(qq, axis=0, keepdims=True)], axis=0).reshape(1, 2, O)

    @pl.when(jnp.logical_and(q % 2 == 0, ib == 0))
    def _():
        s_ref[...] = jnp.zeros_like(s_ref)

    s_ref[...] += part


def _ke_edge(xg, XC, W, C, TW, kp, kr, O):
    def body(xg_ref, xc_ref, w_ref, mx_ref, mn_ref, s_ref):
        q = pl.program_id(0)
        ib = pl.program_id(1)
        xg3 = xg_ref[...].reshape(BN, kp, TW)[:, :, 0:C]
        xc = xc_ref[0][:, 0:C]
        xc3 = jnp.broadcast_to(xc[:, None, :], (BN, kp, C))
        feat = jnp.concatenate([xg3 - xc3, xc3], axis=2)
        out2 = _dot16(feat.reshape(BN * kp, 2 * C), w_ref[...])
        _reduce_write(out2, kp, kr, O, q, ib, mx_ref, mn_ref, s_ref)

    CW = XC.shape[2]
    return pl.pallas_call(
        body,
        grid=(NI, NB),
        in_specs=[
            pl.BlockSpec((BN * kp, TW), lambda q, ib: (NB * q + ib, 0)),
            pl.BlockSpec((1, BN, CW), lambda q, ib: (q, ib, 0)),
            pl.BlockSpec((O, 2 * C), lambda q, ib: (0, 0)),
        ],
        out_specs=[
            pl.BlockSpec((BN, O), lambda q, ib: (NB * q + ib, 0)),
            pl.BlockSpec((BN, O), lambda q, ib: (NB * q + ib, 0)),
            pl.BlockSpec((1, 2, O), lambda q, ib: (q // 2, 0, 0)),
        ],
        out_shape=[
            jax.ShapeDtypeStruct((NT, O), jnp.float32),
            jax.ShapeDtypeStruct((NT, O), jnp.float32),
            jax.ShapeDtypeStruct((2, 2, O), jnp.float32),
        ],
    )(xg, XC, W)


def _ke_orient(xg, X3, P, Wo):
    def body(xg_ref, x3_ref, p_ref, w_ref, mx_ref, mn_ref, s_ref):
        q = pl.program_id(0)
        ib = pl.program_id(1)
        xg3 = xg_ref[...].reshape(BN, KO, 384)
        g3 = xg3[:, :, 0:256]
        zg = xg3[:, :, 256:259]
        ftc = x3_ref[0]
        xtc = p_ref[0]
        ftc3 = jnp.broadcast_to(ftc[:, None, :], (BN, KO, 256))
        xtc3 = jnp.broadcast_to(xtc[:, None, :], (BN, KO, 3))
        feat = jnp.concatenate([g3 - ftc3, g3, zg - xtc3, xtc3], axis=2)
        out2 = _dot16(feat.reshape(BN * KO, 518), w_ref[...])
        _reduce_write(out2, KO, KO, 256, q, ib, mx_ref, mn_ref, s_ref)

    return pl.pallas_call(
        body,
        grid=(NI, NB),
        in_specs=[
            pl.BlockSpec((BN * KO, 384), lambda q, ib: (NB * q + ib, 0)),
            pl.BlockSpec((1, BN, 256), lambda q, ib: ((q + 2) % 4, ib, 0)),
            pl.BlockSpec((1, BN, 3), lambda q, ib: ((q + 2) % 4, ib, 0)),
            pl.BlockSpec((256, 518), lambda q, ib: (0, 0)),
        ],
        out_specs=[
            pl.BlockSpec((BN, 256), lambda q, ib: (NB * q + ib, 0)),
            pl.BlockSpec((BN, 256), lambda q, ib: (NB * q + ib, 0)),
            pl.BlockSpec((1, 2, 256), lambda q, ib: (q // 2, 0, 0)),
        ],
        out_shape=[
            jax.ShapeDtypeStruct((NT, 256), jnp.float32),
            jax.ShapeDtypeStruct((NT, 256), jnp.float32),
            jax.ShapeDtypeStruct((2, 2, 256), jnp.float32),
        ],
    )(xg, X3, P, Wo)


def _bn_max(mx, mn, s_ref, g_ref, b_ref, kr):
    cnt = float(2 * N * kr)
    mean = s_ref[0, 0:1, :] / cnt
    var = s_ref[0, 1:2, :] / cnt - mean * mean
    den = jnp.sqrt(var + EPS)
    gam = g_ref[...]
    bet = b_ref[...]

    def f(v):
        return _leaky(gam * ((v - mean) / den) + bet)

    return jnp.maximum(f(mx), f(mn))


def _kf2(MX, MN, S, gam, bet, kr, O, PW=None, out3d=False):
    def body(mx_ref, mn_ref, s_ref, g_ref, b_ref, o_ref):
        x = _bn_max(mx_ref[...], mn_ref[...], s_ref, g_ref, b_ref, kr)
        if out3d:
            o_ref[0] = x
        elif PW is not None and PW > O:
            o_ref[...] = jnp.concatenate(
                [x, jnp.zeros((N, PW - O), jnp.float32)], axis=1)
        else:
            o_ref[...] = x

    if out3d:
        out_spec = pl.BlockSpec((1, N, O), lambda q: (q, 0, 0))
        out_shape = jax.ShapeDtypeStruct((NI, N, O), jnp.float32)
    else:
        W2 = PW if PW is not None else O
        out_spec = pl.BlockSpec((N, W2), lambda q: (q, 0))
        out_shape = jax.ShapeDtypeStruct((NT, W2), jnp.float32)
    return pl.pallas_call(
        body,
        grid=(NI,),
        in_specs=[
            pl.BlockSpec((N, O), lambda q: (q, 0)),
            pl.BlockSpec((N, O), lambda q: (q, 0)),
            pl.BlockSpec((1, 2, O), lambda q: (q // 2, 0, 0)),
            pl.BlockSpec((1, O), lambda q: (0, 0)),
            pl.BlockSpec((1, O), lambda q: (0, 0)),
        ],
        out_specs=out_spec,
        out_shape=out_shape,
    )(MX, MN, S, gam, bet)


def _k_orient(x3, P):
    def body(xr_ref, xq_ref, pr_ref, idx_ref, tog_ref, d_scr):
        o = pl.program_id(0)
        F = xr_ref[0]
        Q = xq_ref[0]
        nnF = jnp.sum(F * F, axis=1, keepdims=True)
        nnQ = jnp.sum(Q * Q, axis=1, keepdims=True)
        nrow = _diag_row(nnF)
        _score_blocks(d_scr, lambda ib: xr_ref[0, pl.ds(ib * 128, 128), :],
                      Q, nrow, nnQ)
        _topk_store(d_scr, idx_ref, KO, KO, o * N)
        tog_ref[0] = jnp.concatenate(
            [F, pr_ref[0], jnp.zeros((N, 125), jnp.float32)], axis=1)

    return pl.pallas_call(
        body,
        grid=(NI,),
        in_specs=[
            pl.BlockSpec((1, N, 256), lambda o: (o, 0, 0)),
            pl.BlockSpec((1, N, 256), lambda o: ((o + 2) % 4, 0, 0)),
            pl.BlockSpec((1, N, 3), lambda o: (o, 0, 0)),
        ],
        out_specs=[
            pl.BlockSpec((1, N, KO), lambda o: (o, 0, 0)),
            pl.BlockSpec((1, N, 384), lambda o: (o, 0, 0)),
        ],
        out_shape=[
            jax.ShapeDtypeStruct((NI, N, KO), jnp.int32),
            jax.ShapeDtypeStruct((NI, N, 384), jnp.float32),
        ],
        scratch_shapes=[pltpu.VMEM((N, N), jnp.float32)],
    )(x3, x3, P)


def _k_fo(MX, MN, S, gam, bet):
    def body(mx_ref, mn_ref, s_ref, g_ref, b_ref, l0_ref, idx_ref, d_scr):
        q = pl.program_id(0)
        L = _bn_max(mx_ref[...], mn_ref[...], s_ref, g_ref, b_ref, KO)
        l0_ref[0] = L
        nnL = jnp.sum(L * L, axis=1, keepdims=True)
        nrow = _diag_row(nnL)
        _score_blocks(d_scr, lambda ib: l0_ref[0, pl.ds(ib * 128, 128), :],
                      L, nrow, nnL)
        _topk_store(d_scr, idx_ref, KO, KO, q * N)

    return pl.pallas_call(
        body,
        grid=(NI,),
        in_specs=[
            pl.BlockSpec((N, 256), lambda q: (q, 0)),
            pl.BlockSpec((N, 256), lambda q: (q, 0)),
            pl.BlockSpec((1, 2, 256), lambda q: (q // 2, 0, 0)),
            pl.BlockSpec((1, 256), lambda q: (0, 0)),
            pl.BlockSpec((1, 256), lambda q: (0, 0)),
        ],
        out_specs=[
            pl.BlockSpec((1, N, 256), lambda q: (q, 0, 0)),
            pl.BlockSpec((1, N, KO), lambda q: (q, 0, 0)),
        ],
        out_shape=[
            jax.ShapeDtypeStruct((NI, N, 256), jnp.float32),
            jax.ShapeDtypeStruct((NI, N, KO), jnp.int32),
        ],
        scratch_shapes=[pltpu.VMEM((N, N), jnp.float32)],
    )(MX, MN, S, gam, bet)


def _k_final(MX, MN, S, gam, bet, L0):
    def body(mx_ref, mn_ref, s_ref, g_ref, b_ref, l0_ref, o_ref):
        L1 = _bn_max(mx_ref[...], mn_ref[...], s_ref, g_ref, b_ref, KO)
        cat = jnp.concatenate([l0_ref[0], L1], axis=1)
        cmax = jnp.max(cat, axis=0, keepdims=True)
        cmean = jnp.sum(cat, axis=0, keepdims=True) * (1.0 / N)
        o_ref[...] = jnp.concatenate([cmax, cmean], axis=1).reshape(1, 1, 1024)

    return pl.pallas_call(
        body,
        grid=(NI,),
        in_specs=[
            pl.BlockSpec((N, 256), lambda q: (q, 0)),
            pl.BlockSpec((N, 256), lambda q: (q, 0)),
            pl.BlockSpec((1, 2, 256), lambda q: (q // 2, 0, 0)),
            pl.BlockSpec((1, 256), lambda q: (0, 0)),
            pl.BlockSpec((1, 256), lambda q: (0, 0)),
            pl.BlockSpec((1, N, 256), lambda q: (q, 0, 0)),
        ],
        out_specs=pl.BlockSpec((1, 1, 1024), lambda q: (q, 0, 0)),
        out_shape=jax.ShapeDtypeStruct((NI, 1, 1024), jnp.float32),
    )(MX, MN, S, gam, bet, L0)


def kernel(xyz_s, xyz_t, W0, gamma0, beta0, W1, gamma1, beta1, W2, gamma2,
           beta2, Wo, gammao, betao, We, gammae, betae):
    P = jnp.concatenate([xyz_s, xyz_t], axis=0)
    g0, b0 = gamma0.reshape(1, -1), beta0.reshape(1, -1)
    g1, b1 = gamma1.reshape(1, -1), beta1.reshape(1, -1)
    g2, b2 = gamma2.reshape(1, -1), beta2.reshape(1, -1)
    go, bo = gammao.reshape(1, -1), betao.reshape(1, -1)
    ge, be = gammae.reshape(1, -1), betae.reshape(1, -1)

    idxE, T0g = _k1(P)
    idxE_f = idxE.reshape(-1)

    xg0 = _sc_gather(T0g.reshape(NT, 128), idxE_f, KP, 128)
    MX0, MN0, S0 = _ke_edge(xg0, P, W0, 3, 128, KP, KE, 64)
    T1g = _kf2(MX0, MN0, S0, g0, b0, KE, 64, PW=128)

    xg1 = _sc_gather(T1g, idxE_f, KP, 128)
    MX1, MN1, S1 = _ke_edge(xg1, T1g.reshape(NI, N, 128), W1, 64, 128,
                            KP, KE, 128)
    T2g = _kf2(MX1, MN1, S1, g1, b1, KE, 128)

    xg2 = _sc_gather(T2g, idxE_f, KP, 128)
    MX2, MN2, S2 = _ke_edge(xg2, T2g.reshape(NI, N, 128), W2, 128, 128,
                            KP, KE, 256)
    x3 = _kf2(MX2, MN2, S2, g2, b2, KE, 256, out3d=True)

    idxO, Tog = _k_orient(x3, P)
    xgo = _sc_gather(Tog.reshape(NT, 384), idxO.reshape(-1), KO, 384)
    MXo, MNo, So = _ke_orient(xgo, x3, P, Wo)
    L0, idxE2 = _k_fo(MXo, MNo, So, go, bo)

    xge = _sc_gather(L0.reshape(NT, 256), idxE2.reshape(-1), KO, 256)
    MXe, MNe, Se = _ke_edge(xge, L0, We, 256, 256, KO, KO, 256)
    OUT = _k_final(MXe, MNe, Se, ge, be, L0)

    OUT = OUT.reshape(NI, 1024)
    xo = OUT[0:2][:, :, None]
    yo = OUT[2:4][:, :, None]
    return xo, yo


# speedup vs baseline: 1.9159x; 1.9159x over previous
import functools

import jax
import jax.numpy as jnp
from jax import lax
from jax.experimental import pallas as pl
from jax.experimental.pallas import tpu as pltpu
from jax.experimental.pallas import tpu_sc as plsc

N = 1024
NI = 4
NT = NI * N
KE = 27
KP = 28
KO = 24
NB = 8
BN = N // NB
NEG = -3.4e38
EPS = 1e-5


def _leaky(x):
    return jnp.where(x >= 0, x, 0.2 * x)


def _dot16(a, b):
    return lax.dot_general(a.astype(jnp.bfloat16), b.astype(jnp.bfloat16),
                           (((1,), (1,)), ((), ())),
                           preferred_element_type=jnp.float32)


def _diag_row(nn2):
    rowi = lax.broadcasted_iota(jnp.int32, (N, N), 0)
    coli = lax.broadcasted_iota(jnp.int32, (N, N), 1)
    d = jnp.where(rowi == coli, jnp.broadcast_to(nn2, (N, N)), 0.0)
    return jnp.sum(d, axis=0, keepdims=True)


RB = 16


def _topk_store(d_scr, idx_ref, k, k_pad, off):
    cols = lax.broadcasted_iota(jnp.int32, (RB, N), 1)
    tpos = lax.broadcasted_iota(jnp.int32, (RB, k_pad), 1)

    def chunk(i, carry):
        d = d_scr[pl.ds(i * RB, RB), :]
        acc = jnp.zeros((RB, k_pad), jnp.int32)
        first = None
        for t in range(k):
            m = jnp.max(d, axis=1, keepdims=True)
            cand = jnp.where(d >= m, cols, jnp.int32(2 * N))
            am = jnp.min(cand, axis=1, keepdims=True)
            acc = jnp.where(tpos == t, am, acc)
            d = jnp.where(cols == am, NEG, d)
            if t == 0:
                first = am
        if k_pad > k:
            acc = jnp.where(tpos >= k, first, acc)
        idx_ref[0, pl.ds(i * RB, RB), :] = acc + off
        return carry

    lax.fori_loop(0, N // RB, chunk, 0)


def _score_blocks(d_scr, load_row_blk, Full, nn_ref_row, nn_query_col):
    for ib in range(N // 128):
        inner = -2.0 * _dot16(load_row_blk(ib), Full)
        d_scr[pl.ds(ib * 128, 128), :] = (
            (-nn_ref_row) - inner) - nn_query_col[ib * 128:(ib + 1) * 128, :]


def _k1_body(p_ref, idx_ref, t0_ref, d_scr):
    q = pl.program_id(0)
    P = p_ref[0]
    nn2 = jnp.sum(P * P, axis=1, keepdims=True)
    nrow = _diag_row(nn2)
    _score_blocks(d_scr, lambda ib: p_ref[0, pl.ds(ib * 128, 128), :],
                  P, nrow, nn2)
    _topk_store(d_scr, idx_ref, KE, KP, q * N)
    t0_ref[0] = jnp.concatenate([P, jnp.zeros((N, 125), jnp.float32)], axis=1)


def _k1(P):
    return pl.pallas_call(
        _k1_body,
        grid=(NI,),
        in_specs=[pl.BlockSpec((1, N, 3), lambda q: (q, 0, 0))],
        out_specs=[
            pl.BlockSpec((1, N, KP), lambda q: (q, 0, 0)),
            pl.BlockSpec((1, N, 128), lambda q: (q, 0, 0)),
        ],
        out_shape=[
            jax.ShapeDtypeStruct((NI, N, KP), jnp.int32),
            jax.ShapeDtypeStruct((NI, N, 128), jnp.float32),
        ],
        scratch_shapes=[pltpu.VMEM((N, N), jnp.float32)],
    )(P)


def _sc_gather(tab, idx_flat, TW):
    M = idx_flat.shape[0]
    NW = 32
    L = M // NW
    CH = 128
    NCH = L // CH
    mesh = plsc.VectorSubcoreMesh(core_axis_name="c", subcore_axis_name="s")

    @functools.partial(
        pl.kernel,
        mesh=mesh,
        out_type=jax.ShapeDtypeStruct((M, TW), jnp.float32),
        scratch_types=[
            pltpu.VMEM((L,), jnp.int32),
            pltpu.VMEM((CH, TW), jnp.float32),
            pltpu.VMEM((CH, TW), jnp.float32),
            pltpu.SemaphoreType.DMA,
            pltpu.SemaphoreType.DMA,
        ],
    )
    def sc_k(tab_hbm, idx_hbm, out_hbm, idx_v, gb0, gb1, sem0, sem1):
        wid = lax.axis_index("s") * 2 + lax.axis_index("c")
        base = wid * L
        pltpu.sync_copy(idx_hbm.at[pl.ds(base, L)], idx_v)
        bufs = (gb0, gb1)
        sems = (sem0, sem1)

        def fire(c):
            pltpu.async_copy(
                tab_hbm.at[idx_v.at[pl.ds(c * CH, CH)]],
                bufs[c % 2], sems[c % 2])

        fire(0)
        for c in range(NCH):
            if c + 1 < NCH:
                fire(c + 1)
            pltpu.make_async_copy(tab_hbm.at[pl.ds(0, CH)], bufs[c % 2],
                                  sems[c % 2]).wait()
            pltpu.sync_copy(bufs[c % 2], out_hbm.at[pl.ds(base + c * CH, CH)])

    return sc_k(tab, idx_flat)


def _acc_write(vals, O, q, ib, mx_ref, mn_ref, s_ref):
    mx = vals[0]
    mn = vals[0]
    s = vals[0]
    qq = vals[0] * vals[0]
    for v in vals[1:]:
        mx = jnp.maximum(mx, v)
        mn = jnp.minimum(mn, v)
        s = s + v
        qq = qq + v * v
    mx_ref[...] = mx
    mn_ref[...] = mn
    part = jnp.concatenate(
        [jnp.sum(s, axis=0, keepdims=True),
         jnp.sum(qq, axis=0, keepdims=True)], axis=0).reshape(1, 2, O)

    @pl.when(jnp.logical_and(q % 2 == 0, ib == 0))
    def _():
        s_ref[...] = jnp.zeros_like(s_ref)

    s_ref[...] += part


def _ke_edge(xg, XC, W, C, TW, kp, kr, O):
    def body(xg_ref, xc_ref, w_ref, mx_ref, mn_ref, s_ref):
        q = pl.program_id(0)
        ib = pl.program_id(1)
        xc = xc_ref[0][:, 0:C]
        vals = []
        for j in range(kr):
            xgj = xg_ref[j][:, 0:C]
            feat = jnp.concatenate([xgj - xc, xc], axis=1)
            vals.append(_dot16(feat, w_ref[...]))
        _acc_write(vals, O, q, ib, mx_ref, mn_ref, s_ref)

    CW = XC.shape[2]
    return pl.pallas_call(
        body,
        grid=(NI, NB),
        in_specs=[
            pl.BlockSpec((kp, BN, TW), lambda q, ib: (0, NB * q + ib, 0)),
            pl.BlockSpec((1, BN, CW), lambda q, ib: (q, ib, 0)),
            pl.BlockSpec((O, 2 * C), lambda q, ib: (0, 0)),
        ],
        out_specs=[
            pl.BlockSpec((BN, O), lambda q, ib: (NB * q + ib, 0)),
            pl.BlockSpec((BN, O), lambda q, ib: (NB * q + ib, 0)),
            pl.BlockSpec((1, 2, O), lambda q, ib: (q // 2, 0, 0)),
        ],
        out_shape=[
            jax.ShapeDtypeStruct((NT, O), jnp.float32),
            jax.ShapeDtypeStruct((NT, O), jnp.float32),
            jax.ShapeDtypeStruct((2, 2, O), jnp.float32),
        ],
    )(xg, XC, W)


def _ke_orient(xg, X3, P, Wo):
    def body(xg_ref, x3_ref, p_ref, w_ref, mx_ref, mn_ref, s_ref):
        q = pl.program_id(0)
        ib = pl.program_id(1)
        ftc = x3_ref[0]
        xtc = p_ref[0]
        vals = []
        for j in range(KO):
            gj = xg_ref[j][:, 0:256]
            zj = xg_ref[j][:, 256:259]
            feat = jnp.concatenate([gj - ftc, gj, zj - xtc, xtc], axis=1)
            vals.append(_dot16(feat, w_ref[...]))
        _acc_write(vals, 256, q, ib, mx_ref, mn_ref, s_ref)

    return pl.pallas_call(
        body,
        grid=(NI, NB),
        in_specs=[
            pl.BlockSpec((KO, BN, 384), lambda q, ib: (0, NB * q + ib, 0)),
            pl.BlockSpec((1, BN, 256), lambda q, ib: ((q + 2) % 4, ib, 0)),
            pl.BlockSpec((1, BN, 3), lambda q, ib: ((q + 2) % 4, ib, 0)),
            pl.BlockSpec((256, 518), lambda q, ib: (0, 0)),
        ],
        out_specs=[
            pl.BlockSpec((BN, 256), lambda q, ib: (NB * q + ib, 0)),
            pl.BlockSpec((BN, 256), lambda q, ib: (NB * q + ib, 0)),
            pl.BlockSpec((1, 2, 256), lambda q, ib: (q // 2, 0, 0)),
        ],
        out_shape=[
            jax.ShapeDtypeStruct((NT, 256), jnp.float32),
            jax.ShapeDtypeStruct((NT, 256), jnp.float32),
            jax.ShapeDtypeStruct((2, 2, 256), jnp.float32),
        ],
    )(xg, X3, P, Wo)


def _bn_max(mx, mn, s_ref, g_ref, b_ref, kr):
    cnt = float(2 * N * kr)
    mean = s_ref[0, 0:1, :] / cnt
    var = s_ref[0, 1:2, :] / cnt - mean * mean
    den = jnp.sqrt(var + EPS)
    gam = g_ref[...]
    bet = b_ref[...]

    def f(v):
        return _leaky(gam * ((v - mean) / den) + bet)

    return jnp.maximum(f(mx), f(mn))


def _kf2(MX, MN, S, gam, bet, kr, O, PW=None, out3d=False):
    def body(mx_ref, mn_ref, s_ref, g_ref, b_ref, o_ref):
        x = _bn_max(mx_ref[...], mn_ref[...], s_ref, g_ref, b_ref, kr)
        if out3d:
            o_ref[0] = x
        elif PW is not None and PW > O:
            o_ref[...] = jnp.concatenate(
                [x, jnp.zeros((N, PW - O), jnp.float32)], axis=1)
        else:
            o_ref[...] = x

    if out3d:
        out_spec = pl.BlockSpec((1, N, O), lambda q: (q, 0, 0))
        out_shape = jax.ShapeDtypeStruct((NI, N, O), jnp.float32)
    else:
        W2 = PW if PW is not None else O
        out_spec = pl.BlockSpec((N, W2), lambda q: (q, 0))
        out_shape = jax.ShapeDtypeStruct((NT, W2), jnp.float32)
    return pl.pallas_call(
        body,
        grid=(NI,),
        in_specs=[
            pl.BlockSpec((N, O), lambda q: (q, 0)),
            pl.BlockSpec((N, O), lambda q: (q, 0)),
            pl.BlockSpec((1, 2, O), lambda q: (q // 2, 0, 0)),
            pl.BlockSpec((1, O), lambda q: (0, 0)),
            pl.BlockSpec((1, O), lambda q: (0, 0)),
        ],
        out_specs=out_spec,
        out_shape=out_shape,
    )(MX, MN, S, gam, bet)


def _k_orient(x3, P):
    def body(xr_ref, xq_ref, pr_ref, idx_ref, tog_ref, d_scr):
        o = pl.program_id(0)
        F = xr_ref[0]
        Q = xq_ref[0]
        nnF = jnp.sum(F * F, axis=1, keepdims=True)
        nnQ = jnp.sum(Q * Q, axis=1, keepdims=True)
        nrow = _diag_row(nnF)
        _score_blocks(d_scr, lambda ib: xr_ref[0, pl.ds(ib * 128, 128), :],
                      Q, nrow, nnQ)
        _topk_store(d_scr, idx_ref, KO, KO, o * N)
        tog_ref[0] = jnp.concatenate(
            [F, pr_ref[0], jnp.zeros((N, 125), jnp.float32)], axis=1)

    return pl.pallas_call(
        body,
        grid=(NI,),
        in_specs=[
            pl.BlockSpec((1, N, 256), lambda o: (o, 0, 0)),
            pl.BlockSpec((1, N, 256), lambda o: ((o + 2) % 4, 0, 0)),
            pl.BlockSpec((1, N, 3), lambda o: (o, 0, 0)),
        ],
        out_specs=[
            pl.BlockSpec((1, N, KO), lambda o: (o, 0, 0)),
            pl.BlockSpec((1, N, 384), lambda o: (o, 0, 0)),
        ],
        out_shape=[
            jax.ShapeDtypeStruct((NI, N, KO), jnp.int32),
            jax.ShapeDtypeStruct((NI, N, 384), jnp.float32),
        ],
        scratch_shapes=[pltpu.VMEM((N, N), jnp.float32)],
    )(x3, x3, P)


def _k_fo(MX, MN, S, gam, bet):
    def body(mx_ref, mn_ref, s_ref, g_ref, b_ref, l0_ref, idx_ref, d_scr):
        q = pl.program_id(0)
        L = _bn_max(mx_ref[...], mn_ref[...], s_ref, g_ref, b_ref, KO)
        l0_ref[0] = L
        nnL = jnp.sum(L * L, axis=1, keepdims=True)
        nrow = _diag_row(nnL)
        _score_blocks(d_scr, lambda ib: l0_ref[0, pl.ds(ib * 128, 128), :],
                      L, nrow, nnL)
        _topk_store(d_scr, idx_ref, KO, KO, q * N)

    return pl.pallas_call(
        body,
        grid=(NI,),
        in_specs=[
            pl.BlockSpec((N, 256), lambda q: (q, 0)),
            pl.BlockSpec((N, 256), lambda q: (q, 0)),
            pl.BlockSpec((1, 2, 256), lambda q: (q // 2, 0, 0)),
            pl.BlockSpec((1, 256), lambda q: (0, 0)),
            pl.BlockSpec((1, 256), lambda q: (0, 0)),
        ],
        out_specs=[
            pl.BlockSpec((1, N, 256), lambda q: (q, 0, 0)),
            pl.BlockSpec((1, N, KO), lambda q: (q, 0, 0)),
        ],
        out_shape=[
            jax.ShapeDtypeStruct((NI, N, 256), jnp.float32),
            jax.ShapeDtypeStruct((NI, N, KO), jnp.int32),
        ],
        scratch_shapes=[pltpu.VMEM((N, N), jnp.float32)],
    )(MX, MN, S, gam, bet)


def _k_final(MX, MN, S, gam, bet, L0):
    def body(mx_ref, mn_ref, s_ref, g_ref, b_ref, l0_ref, o_ref):
        L1 = _bn_max(mx_ref[...], mn_ref[...], s_ref, g_ref, b_ref, KO)
        cat = jnp.concatenate([l0_ref[0], L1], axis=1)
        cmax = jnp.max(cat, axis=0, keepdims=True)
        cmean = jnp.sum(cat, axis=0, keepdims=True) * (1.0 / N)
        o_ref[...] = jnp.concatenate([cmax, cmean], axis=1).reshape(1, 1, 1024)

    return pl.pallas_call(
        body,
        grid=(NI,),
        in_specs=[
            pl.BlockSpec((N, 256), lambda q: (q, 0)),
            pl.BlockSpec((N, 256), lambda q: (q, 0)),
            pl.BlockSpec((1, 2, 256), lambda q: (q // 2, 0, 0)),
            pl.BlockSpec((1, 256), lambda q: (0, 0)),
            pl.BlockSpec((1, 256), lambda q: (0, 0)),
            pl.BlockSpec((1, N, 256), lambda q: (q, 0, 0)),
        ],
        out_specs=pl.BlockSpec((1, 1, 1024), lambda q: (q, 0, 0)),
        out_shape=jax.ShapeDtypeStruct((NI, 1, 1024), jnp.float32),
    )(MX, MN, S, gam, bet, L0)


def kernel(xyz_s, xyz_t, W0, gamma0, beta0, W1, gamma1, beta1, W2, gamma2,
           beta2, Wo, gammao, betao, We, gammae, betae):
    P = jnp.concatenate([xyz_s, xyz_t], axis=0)
    g0, b0 = gamma0.reshape(1, -1), beta0.reshape(1, -1)
    g1, b1 = gamma1.reshape(1, -1), beta1.reshape(1, -1)
    g2, b2 = gamma2.reshape(1, -1), beta2.reshape(1, -1)
    go, bo = gammao.reshape(1, -1), betao.reshape(1, -1)
    ge, be = gammae.reshape(1, -1), betae.reshape(1, -1)

    idxE, T0g = _k1(P)
    idxE_j = jnp.transpose(idxE[:, :, :KE], (2, 0, 1)).reshape(-1)

    xg0 = _sc_gather(T0g.reshape(NT, 128), idxE_j, 128).reshape(KE, NT, 128)
    MX0, MN0, S0 = _ke_edge(xg0, P, W0, 3, 128, KE, KE, 64)
    T1g = _kf2(MX0, MN0, S0, g0, b0, KE, 64, PW=128)

    xg1 = _sc_gather(T1g, idxE_j, 128).reshape(KE, NT, 128)
    MX1, MN1, S1 = _ke_edge(xg1, T1g.reshape(NI, N, 128), W1, 64, 128,
                            KE, KE, 128)
    T2g = _kf2(MX1, MN1, S1, g1, b1, KE, 128)

    xg2 = _sc_gather(T2g, idxE_j, 128).reshape(KE, NT, 128)
    MX2, MN2, S2 = _ke_edge(xg2, T2g.reshape(NI, N, 128), W2, 128, 128,
                            KE, KE, 256)
    x3 = _kf2(MX2, MN2, S2, g2, b2, KE, 256, out3d=True)

    idxO, Tog = _k_orient(x3, P)
    idxO_j = jnp.transpose(idxO, (2, 0, 1)).reshape(-1)
    xgo = _sc_gather(Tog.reshape(NT, 384), idxO_j, 384).reshape(KO, NT, 384)
    MXo, MNo, So = _ke_orient(xgo, x3, P, Wo)
    L0, idxE2 = _k_fo(MXo, MNo, So, go, bo)

    idxE2_j = jnp.transpose(idxE2, (2, 0, 1)).reshape(-1)
    xge = _sc_gather(L0.reshape(NT, 256), idxE2_j, 256).reshape(KO, NT, 256)
    MXe, MNe, Se = _ke_edge(xge, L0, We, 256, 256, KO, KO, 256)
    OUT = _k_final(MXe, MNe, Se, ge, be, L0)

    OUT = OUT.reshape(NI, 1024)
    xo = OUT[0:2][:, :, None]
    yo = OUT[2:4][:, :, None]
    return xo, yo


# speedup vs baseline: 1.9452x; 1.0153x over previous
import functools

import jax
import jax.numpy as jnp
from jax import lax
from jax.experimental import pallas as pl
from jax.experimental.pallas import tpu as pltpu
from jax.experimental.pallas import tpu_sc as plsc

N = 1024
NI = 4
NT = NI * N
KE = 27
KP = 28
KO = 24
NB = 8
BN = N // NB
NEG = -3.4e38
EPS = 1e-5


def _leaky(x):
    return jnp.where(x >= 0, x, 0.2 * x)


def _dot16(a, b):
    return lax.dot_general(a.astype(jnp.bfloat16), b.astype(jnp.bfloat16),
                           (((1,), (1,)), ((), ())),
                           preferred_element_type=jnp.float32)


def _diag_row(nn2):
    rowi = lax.broadcasted_iota(jnp.int32, (N, N), 0)
    coli = lax.broadcasted_iota(jnp.int32, (N, N), 1)
    d = jnp.where(rowi == coli, jnp.broadcast_to(nn2, (N, N)), 0.0)
    return jnp.sum(d, axis=0, keepdims=True)


RB = 16


def _topk_store(d_scr, idx_ref, k, k_pad, off):
    cols = lax.broadcasted_iota(jnp.int32, (RB, N), 1)
    tpos = lax.broadcasted_iota(jnp.int32, (RB, k_pad), 1)

    def chunk(i, carry):
        d = d_scr[pl.ds(i * RB, RB), :]
        acc = jnp.zeros((RB, k_pad), jnp.int32)
        first = None
        for t in range(k):
            m = jnp.max(d, axis=1, keepdims=True)
            cand = jnp.where(d >= m, cols, jnp.int32(2 * N))
            am = jnp.min(cand, axis=1, keepdims=True)
            acc = jnp.where(tpos == t, am, acc)
            d = jnp.where(cols == am, NEG, d)
            if t == 0:
                first = am
        if k_pad > k:
            acc = jnp.where(tpos >= k, first, acc)
        idx_ref[0, pl.ds(i * RB, RB), :] = acc + off
        return carry

    lax.fori_loop(0, N // RB, chunk, 0)


def _score_blocks(d_scr, load_row_blk, Full, nn_ref_row, nn_query_col):
    for ib in range(N // 128):
        inner = -2.0 * _dot16(load_row_blk(ib), Full)
        d_scr[pl.ds(ib * 128, 128), :] = (
            (-nn_ref_row) - inner) - nn_query_col[ib * 128:(ib + 1) * 128, :]


def _k1_body(p_ref, idx_ref, t0_ref, d_scr):
    q = pl.program_id(0)
    P = p_ref[0]
    nn2 = jnp.sum(P * P, axis=1, keepdims=True)
    nrow = _diag_row(nn2)
    _score_blocks(d_scr, lambda ib: p_ref[0, pl.ds(ib * 128, 128), :],
                  P, nrow, nn2)
    _topk_store(d_scr, idx_ref, KE, KP, q * N)
    t0_ref[0] = jnp.concatenate([P, jnp.zeros((N, 125), jnp.float32)], axis=1)


def _k1(P):
    return pl.pallas_call(
        _k1_body,
        grid=(NI,),
        in_specs=[pl.BlockSpec((1, N, 3), lambda q: (q, 0, 0))],
        out_specs=[
            pl.BlockSpec((1, N, KP), lambda q: (q, 0, 0)),
            pl.BlockSpec((1, N, 128), lambda q: (q, 0, 0)),
        ],
        out_shape=[
            jax.ShapeDtypeStruct((NI, N, KP), jnp.int32),
            jax.ShapeDtypeStruct((NI, N, 128), jnp.float32),
        ],
        scratch_shapes=[pltpu.VMEM((N, N), jnp.float32)],
    )(P)


def _sc_gather(tab, idx_flat, TW):
    M = idx_flat.shape[0]
    NW = 32
    L = M // NW
    CH = 128
    NCH = L // CH
    mesh = plsc.VectorSubcoreMesh(core_axis_name="c", subcore_axis_name="s")

    @functools.partial(
        pl.kernel,
        mesh=mesh,
        out_type=jax.ShapeDtypeStruct((M, TW), jnp.float32),
        scratch_types=[
            pltpu.VMEM((L,), jnp.int32),
            pltpu.VMEM((CH, TW), jnp.float32),
            pltpu.VMEM((CH, TW), jnp.float32),
            pltpu.SemaphoreType.DMA,
            pltpu.SemaphoreType.DMA,
        ],
    )
    def sc_k(tab_hbm, idx_hbm, out_hbm, idx_v, gb0, gb1, sem0, sem1):
        wid = lax.axis_index("s") * 2 + lax.axis_index("c")
        base = wid * L
        pltpu.sync_copy(idx_hbm.at[pl.ds(base, L)], idx_v)
        bufs = (gb0, gb1)
        sems = (sem0, sem1)

        def fire(c):
            pltpu.async_copy(
                tab_hbm.at[idx_v.at[pl.ds(c * CH, CH)]],
                bufs[c % 2], sems[c % 2])

        fire(0)
        for c in range(NCH):
            if c + 1 < NCH:
                fire(c + 1)
            pltpu.make_async_copy(tab_hbm.at[pl.ds(0, CH)], bufs[c % 2],
                                  sems[c % 2]).wait()
            pltpu.sync_copy(bufs[c % 2], out_hbm.at[pl.ds(base + c * CH, CH)])

    return sc_k(tab, idx_flat)


def _acc_write(vals, O, q, ib, mx_ref, mn_ref, s_ref):
    mx = vals[0]
    mn = vals[0]
    s = vals[0]
    qq = vals[0] * vals[0]
    for v in vals[1:]:
        mx = jnp.maximum(mx, v)
        mn = jnp.minimum(mn, v)
        s = s + v
        qq = qq + v * v
    mx_ref[...] = mx
    mn_ref[...] = mn
    part = jnp.concatenate(
        [jnp.sum(s, axis=0, keepdims=True),
         jnp.sum(qq, axis=0, keepdims=True)], axis=0).reshape(1, 2, O)

    @pl.when(jnp.logical_and(q % 2 == 0, ib == 0))
    def _():
        s_ref[...] = jnp.zeros_like(s_ref)

    s_ref[...] += part


def _ke_edge(xg, XC, W, C, TW, kp, kr, O):
    def body(xg_ref, xc_ref, w_ref, mx_ref, mn_ref, s_ref):
        q = pl.program_id(0)
        ib = pl.program_id(1)
        xc = xc_ref[0][:, 0:C]
        feats = [jnp.concatenate([xg_ref[j][:, 0:C] - xc, xc], axis=1)
                 for j in range(kr)]
        out = _dot16(jnp.concatenate(feats, axis=0), w_ref[...])
        vals = [out[j * BN:(j + 1) * BN] for j in range(kr)]
        _acc_write(vals, O, q, ib, mx_ref, mn_ref, s_ref)

    CW = XC.shape[2]
    return pl.pallas_call(
        body,
        grid=(NI, NB),
        in_specs=[
            pl.BlockSpec((kp, BN, TW), lambda q, ib: (0, NB * q + ib, 0)),
            pl.BlockSpec((1, BN, CW), lambda q, ib: (q, ib, 0)),
            pl.BlockSpec((O, 2 * C), lambda q, ib: (0, 0)),
        ],
        out_specs=[
            pl.BlockSpec((BN, O), lambda q, ib: (NB * q + ib, 0)),
            pl.BlockSpec((BN, O), lambda q, ib: (NB * q + ib, 0)),
            pl.BlockSpec((1, 2, O), lambda q, ib: (q // 2, 0, 0)),
        ],
        out_shape=[
            jax.ShapeDtypeStruct((NT, O), jnp.float32),
            jax.ShapeDtypeStruct((NT, O), jnp.float32),
            jax.ShapeDtypeStruct((2, 2, O), jnp.float32),
        ],
    )(xg, XC, W)


def _ke_orient(xg, X3, P, Wo):
    def body(xg_ref, x3_ref, p_ref, w_ref, mx_ref, mn_ref, s_ref):
        q = pl.program_id(0)
        ib = pl.program_id(1)
        ftc = x3_ref[0]
        xtc = p_ref[0]
        feats = []
        for j in range(KO):
            gj = xg_ref[j][:, 0:256]
            zj = xg_ref[j][:, 256:259]
            feats.append(
                jnp.concatenate([gj - ftc, gj, zj - xtc, xtc], axis=1))
        out = _dot16(jnp.concatenate(feats, axis=0), w_ref[...])
        vals = [out[j * BN:(j + 1) * BN] for j in range(KO)]
        _acc_write(vals, 256, q, ib, mx_ref, mn_ref, s_ref)

    return pl.pallas_call(
        body,
        grid=(NI, NB),
        in_specs=[
            pl.BlockSpec((KO, BN, 384), lambda q, ib: (0, NB * q + ib, 0)),
            pl.BlockSpec((1, BN, 256), lambda q, ib: ((q + 2) % 4, ib, 0)),
            pl.BlockSpec((1, BN, 3), lambda q, ib: ((q + 2) % 4, ib, 0)),
            pl.BlockSpec((256, 518), lambda q, ib: (0, 0)),
        ],
        out_specs=[
            pl.BlockSpec((BN, 256), lambda q, ib: (NB * q + ib, 0)),
            pl.BlockSpec((BN, 256), lambda q, ib: (NB * q + ib, 0)),
            pl.BlockSpec((1, 2, 256), lambda q, ib: (q // 2, 0, 0)),
        ],
        out_shape=[
            jax.ShapeDtypeStruct((NT, 256), jnp.float32),
            jax.ShapeDtypeStruct((NT, 256), jnp.float32),
            jax.ShapeDtypeStruct((2, 2, 256), jnp.float32),
        ],
    )(xg, X3, P, Wo)


def _bn_max(mx, mn, s_ref, g_ref, b_ref, kr):
    cnt = float(2 * N * kr)
    mean = s_ref[0, 0:1, :] / cnt
    var = s_ref[0, 1:2, :] / cnt - mean * mean
    den = jnp.sqrt(var + EPS)
    gam = g_ref[...]
    bet = b_ref[...]

    def f(v):
        return _leaky(gam * ((v - mean) / den) + bet)

    return jnp.maximum(f(mx), f(mn))


def _kf2(MX, MN, S, gam, bet, kr, O, PW=None, out3d=False):
    def body(mx_ref, mn_ref, s_ref, g_ref, b_ref, o_ref):
        x = _bn_max(mx_ref[...], mn_ref[...], s_ref, g_ref, b_ref, kr)
        if out3d:
            o_ref[0] = x
        elif PW is not None and PW > O:
            o_ref[...] = jnp.concatenate(
                [x, jnp.zeros((N, PW - O), jnp.float32)], axis=1)
        else:
            o_ref[...] = x

    if out3d:
        out_spec = pl.BlockSpec((1, N, O), lambda q: (q, 0, 0))
        out_shape = jax.ShapeDtypeStruct((NI, N, O), jnp.float32)
    else:
        W2 = PW if PW is not None else O
        out_spec = pl.BlockSpec((N, W2), lambda q: (q, 0))
        out_shape = jax.ShapeDtypeStruct((NT, W2), jnp.float32)
    return pl.pallas_call(
        body,
        grid=(NI,),
        in_specs=[
            pl.BlockSpec((N, O), lambda q: (q, 0)),
            pl.BlockSpec((N, O), lambda q: (q, 0)),
            pl.BlockSpec((1, 2, O), lambda q: (q // 2, 0, 0)),
            pl.BlockSpec((1, O), lambda q: (0, 0)),
            pl.BlockSpec((1, O), lambda q: (0, 0)),
        ],
        out_specs=out_spec,
        out_shape=out_shape,
    )(MX, MN, S, gam, bet)


def _k_orient(x3, P):
    def body(xr_ref, xq_ref, pr_ref, idx_ref, tog_ref, d_scr):
        o = pl.program_id(0)
        F = xr_ref[0]
        Q = xq_ref[0]
        nnF = jnp.sum(F * F, axis=1, keepdims=True)
        nnQ = jnp.sum(Q * Q, axis=1, keepdims=True)
        nrow = _diag_row(nnF)
        _score_blocks(d_scr, lambda ib: xr_ref[0, pl.ds(ib * 128, 128), :],
                      Q, nrow, nnQ)
        _topk_store(d_scr, idx_ref, KO, KO, o * N)
        tog_ref[0] = jnp.concatenate(
            [F, pr_ref[0], jnp.zeros((N, 125), jnp.float32)], axis=1)

    return pl.pallas_call(
        body,
        grid=(NI,),
        in_specs=[
            pl.BlockSpec((1, N, 256), lambda o: (o, 0, 0)),
            pl.BlockSpec((1, N, 256), lambda o: ((o + 2) % 4, 0, 0)),
            pl.BlockSpec((1, N, 3), lambda o: (o, 0, 0)),
        ],
        out_specs=[
            pl.BlockSpec((1, N, KO), lambda o: (o, 0, 0)),
            pl.BlockSpec((1, N, 384), lambda o: (o, 0, 0)),
        ],
        out_shape=[
            jax.ShapeDtypeStruct((NI, N, KO), jnp.int32),
            jax.ShapeDtypeStruct((NI, N, 384), jnp.float32),
        ],
        scratch_shapes=[pltpu.VMEM((N, N), jnp.float32)],
    )(x3, x3, P)


def _k_fo(MX, MN, S, gam, bet):
    def body(mx_ref, mn_ref, s_ref, g_ref, b_ref, l0_ref, idx_ref, d_scr):
        q = pl.program_id(0)
        L = _bn_max(mx_ref[...], mn_ref[...], s_ref, g_ref, b_ref, KO)
        l0_ref[0] = L
        nnL = jnp.sum(L * L, axis=1, keepdims=True)
        nrow = _diag_row(nnL)
        _score_blocks(d_scr, lambda ib: l0_ref[0, pl.ds(ib * 128, 128), :],
                      L, nrow, nnL)
        _topk_store(d_scr, idx_ref, KO, KO, q * N)

    return pl.pallas_call(
        body,
        grid=(NI,),
        in_specs=[
            pl.BlockSpec((N, 256), lambda q: (q, 0)),
            pl.BlockSpec((N, 256), lambda q: (q, 0)),
            pl.BlockSpec((1, 2, 256), lambda q: (q // 2, 0, 0)),
            pl.BlockSpec((1, 256), lambda q: (0, 0)),
            pl.BlockSpec((1, 256), lambda q: (0, 0)),
        ],
        out_specs=[
            pl.BlockSpec((1, N, 256), lambda q: (q, 0, 0)),
            pl.BlockSpec((1, N, KO), lambda q: (q, 0, 0)),
        ],
        out_shape=[
            jax.ShapeDtypeStruct((NI, N, 256), jnp.float32),
            jax.ShapeDtypeStruct((NI, N, KO), jnp.int32),
        ],
        scratch_shapes=[pltpu.VMEM((N, N), jnp.float32)],
    )(MX, MN, S, gam, bet)


def _k_final(MX, MN, S, gam, bet, L0):
    def body(mx_ref, mn_ref, s_ref, g_ref, b_ref, l0_ref, o_ref):
        L1 = _bn_max(mx_ref[...], mn_ref[...], s_ref, g_ref, b_ref, KO)
        cat = jnp.concatenate([l0_ref[0], L1], axis=1)
        cmax = jnp.max(cat, axis=0, keepdims=True)
        cmean = jnp.sum(cat, axis=0, keepdims=True) * (1.0 / N)
        o_ref[...] = jnp.concatenate([cmax, cmean], axis=1).reshape(1, 1, 1024)

    return pl.pallas_call(
        body,
        grid=(NI,),
        in_specs=[
            pl.BlockSpec((N, 256), lambda q: (q, 0)),
            pl.BlockSpec((N, 256), lambda q: (q, 0)),
            pl.BlockSpec((1, 2, 256), lambda q: (q // 2, 0, 0)),
            pl.BlockSpec((1, 256), lambda q: (0, 0)),
            pl.BlockSpec((1, 256), lambda q: (0, 0)),
            pl.BlockSpec((1, N, 256), lambda q: (q, 0, 0)),
        ],
        out_specs=pl.BlockSpec((1, 1, 1024), lambda q: (q, 0, 0)),
        out_shape=jax.ShapeDtypeStruct((NI, 1, 1024), jnp.float32),
    )(MX, MN, S, gam, bet, L0)


def kernel(xyz_s, xyz_t, W0, gamma0, beta0, W1, gamma1, beta1, W2, gamma2,
           beta2, Wo, gammao, betao, We, gammae, betae):
    P = jnp.concatenate([xyz_s, xyz_t], axis=0)
    g0, b0 = gamma0.reshape(1, -1), beta0.reshape(1, -1)
    g1, b1 = gamma1.reshape(1, -1), beta1.reshape(1, -1)
    g2, b2 = gamma2.reshape(1, -1), beta2.reshape(1, -1)
    go, bo = gammao.reshape(1, -1), betao.reshape(1, -1)
    ge, be = gammae.reshape(1, -1), betae.reshape(1, -1)

    idxE, T0g = _k1(P)
    idxE_j = jnp.transpose(idxE[:, :, :KE], (2, 0, 1)).reshape(-1)

    xg0 = _sc_gather(T0g.reshape(NT, 128), idxE_j, 128).reshape(KE, NT, 128)
    MX0, MN0, S0 = _ke_edge(xg0, P, W0, 3, 128, KE, KE, 64)
    T1g = _kf2(MX0, MN0, S0, g0, b0, KE, 64, PW=128)

    xg1 = _sc_gather(T1g, idxE_j, 128).reshape(KE, NT, 128)
    MX1, MN1, S1 = _ke_edge(xg1, T1g.reshape(NI, N, 128), W1, 64, 128,
                            KE, KE, 128)
    T2g = _kf2(MX1, MN1, S1, g1, b1, KE, 128)

    xg2 = _sc_gather(T2g, idxE_j, 128).reshape(KE, NT, 128)
    MX2, MN2, S2 = _ke_edge(xg2, T2g.reshape(NI, N, 128), W2, 128, 128,
                            KE, KE, 256)
    x3 = _kf2(MX2, MN2, S2, g2, b2, KE, 256, out3d=True)

    idxO, Tog = _k_orient(x3, P)
    idxO_j = jnp.transpose(idxO, (2, 0, 1)).reshape(-1)
    xgo = _sc_gather(Tog.reshape(NT, 384), idxO_j, 384).reshape(KO, NT, 384)
    MXo, MNo, So = _ke_orient(xgo, x3, P, Wo)
    L0, idxE2 = _k_fo(MXo, MNo, So, go, bo)

    idxE2_j = jnp.transpose(idxE2, (2, 0, 1)).reshape(-1)
    xge = _sc_gather(L0.reshape(NT, 256), idxE2_j, 256).reshape(KO, NT, 256)
    MXe, MNe, Se = _ke_edge(xge, L0, We, 256, 256, KO, KO, 256)
    OUT = _k_final(MXe, MNe, Se, ge, be, L0)

    OUT = OUT.reshape(NI, 1024)
    xo = OUT[0:2][:, :, None]
    yo = OUT[2:4][:, :, None]
    return xo, yo


# speedup vs baseline: 3.3343x; 1.7141x over previous
import functools

import jax
import jax.numpy as jnp
from jax import lax
from jax.experimental import pallas as pl
from jax.experimental.pallas import tpu as pltpu
from jax.experimental.pallas import tpu_sc as plsc

N = 1024
NI = 4
NT = NI * N
KE = 27
KP = 28
KO = 24
NB = 8
BN = N // NB
NEG = -3.4e38
EPS = 1e-5


def _leaky(x):
    return jnp.where(x >= 0, x, 0.2 * x)


def _dot16(a, b):
    return lax.dot_general(a.astype(jnp.bfloat16), b.astype(jnp.bfloat16),
                           (((1,), (1,)), ((), ())),
                           preferred_element_type=jnp.float32)


def _diag_row(nn2):
    rowi = lax.broadcasted_iota(jnp.int32, (N, N), 0)
    coli = lax.broadcasted_iota(jnp.int32, (N, N), 1)
    d = jnp.where(rowi == coli, jnp.broadcast_to(nn2, (N, N)), 0.0)
    return jnp.sum(d, axis=0, keepdims=True)


RB = 32


def _topk_store(d_scr, idx_ref, k, k_pad, off):
    cols = lax.broadcasted_iota(jnp.int32, (RB, N), 1)
    tpos = lax.broadcasted_iota(jnp.int32, (RB, k_pad), 1)

    def chunk(i, carry):
        d = d_scr[pl.ds(i * RB, RB), :]
        acc = jnp.zeros((RB, k_pad), jnp.int32)
        first = None
        for t in range(k):
            m = jnp.max(d, axis=1, keepdims=True)
            cand = jnp.where(d >= m, cols, jnp.int32(2 * N))
            am = jnp.min(cand, axis=1, keepdims=True)
            acc = jnp.where(tpos == t, am, acc)
            d = jnp.where(cols == am, NEG, d)
            if t == 0:
                first = am
        if k_pad > k:
            acc = jnp.where(tpos >= k, first, acc)
        idx_ref[0, pl.ds(i * RB, RB), :] = acc + off
        return carry

    lax.fori_loop(0, N // RB, chunk, 0)


def _score_blocks(d_scr, load_row_blk, Full, nn_ref_row, nn_query_col):
    for ib in range(N // 128):
        inner = -2.0 * _dot16(load_row_blk(ib), Full)
        d_scr[pl.ds(ib * 128, 128), :] = (
            (-nn_ref_row) - inner) - nn_query_col[ib * 128:(ib + 1) * 128, :]


def _k1_body(p_ref, idx_ref, t0_ref, d_scr):
    q = pl.program_id(0)
    P = p_ref[0]
    nn2 = jnp.sum(P * P, axis=1, keepdims=True)
    nrow = _diag_row(nn2)
    _score_blocks(d_scr, lambda ib: p_ref[0, pl.ds(ib * 128, 128), :],
                  P, nrow, nn2)
    _topk_store(d_scr, idx_ref, KE, KP, q * N)
    t0_ref[0] = jnp.concatenate([P, jnp.zeros((N, 125), jnp.float32)], axis=1)


def _k1(P):
    return pl.pallas_call(
        _k1_body,
        grid=(NI,),
        in_specs=[pl.BlockSpec((1, N, 3), lambda q: (q, 0, 0))],
        out_specs=[
            pl.BlockSpec((1, N, KP), lambda q: (q, 0, 0)),
            pl.BlockSpec((1, N, 128), lambda q: (q, 0, 0)),
        ],
        out_shape=[
            jax.ShapeDtypeStruct((NI, N, KP), jnp.int32),
            jax.ShapeDtypeStruct((NI, N, 128), jnp.float32),
        ],
        scratch_shapes=[pltpu.VMEM((N, N), jnp.float32)],
    )(P)


def _sc_gather(tab, idx_flat, TW):
    M = idx_flat.shape[0]
    NW = 32
    L = M // NW
    CH = 128
    NCH = L // CH
    mesh = plsc.VectorSubcoreMesh(core_axis_name="c", subcore_axis_name="s")

    @functools.partial(
        pl.kernel,
        mesh=mesh,
        out_type=jax.ShapeDtypeStruct((M, TW), jnp.float32),
        scratch_types=[
            pltpu.VMEM((L,), jnp.int32),
            pltpu.VMEM((CH, TW), jnp.float32),
            pltpu.VMEM((CH, TW), jnp.float32),
            pltpu.SemaphoreType.DMA,
            pltpu.SemaphoreType.DMA,
        ],
    )
    def sc_k(tab_hbm, idx_hbm, out_hbm, idx_v, gb0, gb1, sem0, sem1):
        wid = lax.axis_index("s") * 2 + lax.axis_index("c")
        base = wid * L
        pltpu.sync_copy(idx_hbm.at[pl.ds(base, L)], idx_v)
        bufs = (gb0, gb1)
        sems = (sem0, sem1)

        def fire(c):
            pltpu.async_copy(
                tab_hbm.at[idx_v.at[pl.ds(c * CH, CH)]],
                bufs[c % 2], sems[c % 2])

        fire(0)
        for c in range(NCH):
            if c + 1 < NCH:
                fire(c + 1)
            pltpu.make_async_copy(tab_hbm.at[pl.ds(0, CH)], bufs[c % 2],
                                  sems[c % 2]).wait()
            pltpu.sync_copy(bufs[c % 2], out_hbm.at[pl.ds(base + c * CH, CH)])

    return sc_k(tab, idx_flat)


def _acc_write(vals, O, q, ib, mx_ref, mn_ref, s_ref):
    mx = vals[0]
    mn = vals[0]
    s = vals[0]
    qq = vals[0] * vals[0]
    for v in vals[1:]:
        mx = jnp.maximum(mx, v)
        mn = jnp.minimum(mn, v)
        s = s + v
        qq = qq + v * v
    mx_ref[...] = mx
    mn_ref[...] = mn
    part = jnp.concatenate(
        [jnp.sum(s, axis=0, keepdims=True),
         jnp.sum(qq, axis=0, keepdims=True)], axis=0).reshape(1, 2, O)

    @pl.when(jnp.logical_and(q % 2 == 0, ib == 0))
    def _():
        s_ref[...] = jnp.zeros_like(s_ref)

    s_ref[...] += part


def _ke_edge(xg, XC, W, C, TW, kp, kr, O):
    def body(xg_ref, xc_ref, w_ref, mx_ref, mn_ref, s_ref):
        q = pl.program_id(0)
        ib = pl.program_id(1)
        xc = xc_ref[0][:, 0:C]
        feats = [jnp.concatenate([xg_ref[j][:, 0:C] - xc, xc], axis=1)
                 for j in range(kr)]
        out = _dot16(jnp.concatenate(feats, axis=0), w_ref[...])
        vals = [out[j * BN:(j + 1) * BN] for j in range(kr)]
        _acc_write(vals, O, q, ib, mx_ref, mn_ref, s_ref)

    CW = XC.shape[2]
    return pl.pallas_call(
        body,
        grid=(NI, NB),
        in_specs=[
            pl.BlockSpec((kp, BN, TW), lambda q, ib: (0, NB * q + ib, 0)),
            pl.BlockSpec((1, BN, CW), lambda q, ib: (q, ib, 0)),
            pl.BlockSpec((O, 2 * C), lambda q, ib: (0, 0)),
        ],
        out_specs=[
            pl.BlockSpec((BN, O), lambda q, ib: (NB * q + ib, 0)),
            pl.BlockSpec((BN, O), lambda q, ib: (NB * q + ib, 0)),
            pl.BlockSpec((1, 2, O), lambda q, ib: (q // 2, 0, 0)),
        ],
        out_shape=[
            jax.ShapeDtypeStruct((NT, O), jnp.float32),
            jax.ShapeDtypeStruct((NT, O), jnp.float32),
            jax.ShapeDtypeStruct((2, 2, O), jnp.float32),
        ],
    )(xg, XC, W)


def _ke_orient(xg, X3, P, Wo):
    def body(xg_ref, x3_ref, p_ref, w_ref, mx_ref, mn_ref, s_ref):
        q = pl.program_id(0)
        ib = pl.program_id(1)
        ftc = x3_ref[0]
        xtc = p_ref[0]
        feats = []
        for j in range(KO):
            gj = xg_ref[j][:, 0:256]
            zj = xg_ref[j][:, 256:259]
            feats.append(
                jnp.concatenate([gj - ftc, gj, zj - xtc, xtc], axis=1))
        out = _dot16(jnp.concatenate(feats, axis=0), w_ref[...])
        vals = [out[j * BN:(j + 1) * BN] for j in range(KO)]
        _acc_write(vals, 256, q, ib, mx_ref, mn_ref, s_ref)

    return pl.pallas_call(
        body,
        grid=(NI, NB),
        in_specs=[
            pl.BlockSpec((KO, BN, 384), lambda q, ib: (0, NB * q + ib, 0)),
            pl.BlockSpec((1, BN, 256), lambda q, ib: ((q + 2) % 4, ib, 0)),
            pl.BlockSpec((1, BN, 3), lambda q, ib: ((q + 2) % 4, ib, 0)),
            pl.BlockSpec((256, 518), lambda q, ib: (0, 0)),
        ],
        out_specs=[
            pl.BlockSpec((BN, 256), lambda q, ib: (NB * q + ib, 0)),
            pl.BlockSpec((BN, 256), lambda q, ib: (NB * q + ib, 0)),
            pl.BlockSpec((1, 2, 256), lambda q, ib: (q // 2, 0, 0)),
        ],
        out_shape=[
            jax.ShapeDtypeStruct((NT, 256), jnp.float32),
            jax.ShapeDtypeStruct((NT, 256), jnp.float32),
            jax.ShapeDtypeStruct((2, 2, 256), jnp.float32),
        ],
    )(xg, X3, P, Wo)


def _bn_max(mx, mn, s_ref, g_ref, b_ref, kr):
    cnt = float(2 * N * kr)
    mean = s_ref[0, 0:1, :] / cnt
    var = s_ref[0, 1:2, :] / cnt - mean * mean
    den = jnp.sqrt(var + EPS)
    gam = g_ref[...]
    bet = b_ref[...]

    def f(v):
        return _leaky(gam * ((v - mean) / den) + bet)

    return jnp.maximum(f(mx), f(mn))


def _kf2(MX, MN, S, gam, bet, kr, O, PW=None, out3d=False):
    def body(mx_ref, mn_ref, s_ref, g_ref, b_ref, o_ref):
        x = _bn_max(mx_ref[...], mn_ref[...], s_ref, g_ref, b_ref, kr)
        if out3d:
            o_ref[0] = x
        elif PW is not None and PW > O:
            o_ref[...] = jnp.concatenate(
                [x, jnp.zeros((N, PW - O), jnp.float32)], axis=1)
        else:
            o_ref[...] = x

    if out3d:
        out_spec = pl.BlockSpec((1, N, O), lambda q: (q, 0, 0))
        out_shape = jax.ShapeDtypeStruct((NI, N, O), jnp.float32)
    else:
        W2 = PW if PW is not None else O
        out_spec = pl.BlockSpec((N, W2), lambda q: (q, 0))
        out_shape = jax.ShapeDtypeStruct((NT, W2), jnp.float32)
    return pl.pallas_call(
        body,
        grid=(NI,),
        in_specs=[
            pl.BlockSpec((N, O), lambda q: (q, 0)),
            pl.BlockSpec((N, O), lambda q: (q, 0)),
            pl.BlockSpec((1, 2, O), lambda q: (q // 2, 0, 0)),
            pl.BlockSpec((1, O), lambda q: (0, 0)),
            pl.BlockSpec((1, O), lambda q: (0, 0)),
        ],
        out_specs=out_spec,
        out_shape=out_shape,
    )(MX, MN, S, gam, bet)


def _k_orient(x3, P):
    def body(xr_ref, xq_ref, pr_ref, idx_ref, tog_ref, d_scr):
        o = pl.program_id(0)
        F = xr_ref[0]
        Q = xq_ref[0]
        nnF = jnp.sum(F * F, axis=1, keepdims=True)
        nnQ = jnp.sum(Q * Q, axis=1, keepdims=True)
        nrow = _diag_row(nnF)
        _score_blocks(d_scr, lambda ib: xr_ref[0, pl.ds(ib * 128, 128), :],
                      Q, nrow, nnQ)
        _topk_store(d_scr, idx_ref, KO, KO, o * N)
        tog_ref[0] = jnp.concatenate(
            [F, pr_ref[0], jnp.zeros((N, 125), jnp.float32)], axis=1)

    return pl.pallas_call(
        body,
        grid=(NI,),
        in_specs=[
            pl.BlockSpec((1, N, 256), lambda o: (o, 0, 0)),
            pl.BlockSpec((1, N, 256), lambda o: ((o + 2) % 4, 0, 0)),
            pl.BlockSpec((1, N, 3), lambda o: (o, 0, 0)),
        ],
        out_specs=[
            pl.BlockSpec((1, N, KO), lambda o: (o, 0, 0)),
            pl.BlockSpec((1, N, 384), lambda o: (o, 0, 0)),
        ],
        out_shape=[
            jax.ShapeDtypeStruct((NI, N, KO), jnp.int32),
            jax.ShapeDtypeStruct((NI, N, 384), jnp.float32),
        ],
        scratch_shapes=[pltpu.VMEM((N, N), jnp.float32)],
    )(x3, x3, P)


def _k_fo(MX, MN, S, gam, bet):
    def body(mx_ref, mn_ref, s_ref, g_ref, b_ref, l0_ref, idx_ref, d_scr):
        q = pl.program_id(0)
        L = _bn_max(mx_ref[...], mn_ref[...], s_ref, g_ref, b_ref, KO)
        l0_ref[0] = L
        nnL = jnp.sum(L * L, axis=1, keepdims=True)
        nrow = _diag_row(nnL)
        _score_blocks(d_scr, lambda ib: l0_ref[0, pl.ds(ib * 128, 128), :],
                      L, nrow, nnL)
        _topk_store(d_scr, idx_ref, KO, KO, q * N)

    return pl.pallas_call(
        body,
        grid=(NI,),
        in_specs=[
            pl.BlockSpec((N, 256), lambda q: (q, 0)),
            pl.BlockSpec((N, 256), lambda q: (q, 0)),
            pl.BlockSpec((1, 2, 256), lambda q: (q // 2, 0, 0)),
            pl.BlockSpec((1, 256), lambda q: (0, 0)),
            pl.BlockSpec((1, 256), lambda q: (0, 0)),
        ],
        out_specs=[
            pl.BlockSpec((1, N, 256), lambda q: (q, 0, 0)),
            pl.BlockSpec((1, N, KO), lambda q: (q, 0, 0)),
        ],
        out_shape=[
            jax.ShapeDtypeStruct((NI, N, 256), jnp.float32),
            jax.ShapeDtypeStruct((NI, N, KO), jnp.int32),
        ],
        scratch_shapes=[pltpu.VMEM((N, N), jnp.float32)],
    )(MX, MN, S, gam, bet)


def _k_final(MX, MN, S, gam, bet, L0):
    def body(mx_ref, mn_ref, s_ref, g_ref, b_ref, l0_ref, o_ref):
        L1 = _bn_max(mx_ref[...], mn_ref[...], s_ref, g_ref, b_ref, KO)
        cat = jnp.concatenate([l0_ref[0], L1], axis=1)
        cmax = jnp.max(cat, axis=0, keepdims=True)
        cmean = jnp.sum(cat, axis=0, keepdims=True) * (1.0 / N)
        o_ref[...] = jnp.concatenate([cmax, cmean], axis=1).reshape(1, 1, 1024)

    return pl.pallas_call(
        body,
        grid=(NI,),
        in_specs=[
            pl.BlockSpec((N, 256), lambda q: (q, 0)),
            pl.BlockSpec((N, 256), lambda q: (q, 0)),
            pl.BlockSpec((1, 2, 256), lambda q: (q // 2, 0, 0)),
            pl.BlockSpec((1, 256), lambda q: (0, 0)),
            pl.BlockSpec((1, 256), lambda q: (0, 0)),
            pl.BlockSpec((1, N, 256), lambda q: (q, 0, 0)),
        ],
        out_specs=pl.BlockSpec((1, 1, 1024), lambda q: (q, 0, 0)),
        out_shape=jax.ShapeDtypeStruct((NI, 1, 1024), jnp.float32),
    )(MX, MN, S, gam, bet, L0)


def kernel(xyz_s, xyz_t, W0, gamma0, beta0, W1, gamma1, beta1, W2, gamma2,
           beta2, Wo, gammao, betao, We, gammae, betae):
    P = jnp.concatenate([xyz_s, xyz_t], axis=0)
    g0, b0 = gamma0.reshape(1, -1), beta0.reshape(1, -1)
    g1, b1 = gamma1.reshape(1, -1), beta1.reshape(1, -1)
    g2, b2 = gamma2.reshape(1, -1), beta2.reshape(1, -1)
    go, bo = gammao.reshape(1, -1), betao.reshape(1, -1)
    ge, be = gammae.reshape(1, -1), betae.reshape(1, -1)

    idxE, T0g = _k1(P)
    idxE_j = jnp.transpose(idxE[:, :, :KE], (2, 0, 1)).reshape(-1)

    xg0 = _sc_gather(T0g.reshape(NT, 128), idxE_j, 128).reshape(KE, NT, 128)
    MX0, MN0, S0 = _ke_edge(xg0, P, W0, 3, 128, KE, KE, 64)
    T1g = _kf2(MX0, MN0, S0, g0, b0, KE, 64, PW=128)

    xg1 = _sc_gather(T1g, idxE_j, 128).reshape(KE, NT, 128)
    MX1, MN1, S1 = _ke_edge(xg1, T1g.reshape(NI, N, 128), W1, 64, 128,
                            KE, KE, 128)
    T2g = _kf2(MX1, MN1, S1, g1, b1, KE, 128)

    xg2 = _sc_gather(T2g, idxE_j, 128).reshape(KE, NT, 128)
    MX2, MN2, S2 = _ke_edge(xg2, T2g.reshape(NI, N, 128), W2, 128, 128,
                            KE, KE, 256)
    x3 = _kf2(MX2, MN2, S2, g2, b2, KE, 256, out3d=True)

    idxO, Tog = _k_orient(x3, P)
    idxO_j = jnp.transpose(idxO, (2, 0, 1)).reshape(-1)
    xgo = _sc_gather(Tog.reshape(NT, 384), idxO_j, 384).reshape(KO, NT, 384)
    MXo, MNo, So = _ke_orient(xgo, x3, P, Wo)
    L0, idxE2 = _k_fo(MXo, MNo, So, go, bo)

    idxE2_j = jnp.transpose(idxE2, (2, 0, 1)).reshape(-1)
    xge = _sc_gather(L0.reshape(NT, 256), idxE2_j, 256).reshape(KO, NT, 256)
    MXe, MNe, Se = _ke_edge(xge, L0, We, 256, 256, KO, KO, 256)
    OUT = _k_final(MXe, MNe, Se, ge, be, L0)

    OUT = OUT.reshape(NI, 1024)
    xo = OUT[0:2][:, :, None]
    yo = OUT[2:4][:, :, None]
    return xo, yo


# speedup vs baseline: 5.3090x; 1.5923x over previous
import functools

import jax
import jax.numpy as jnp
from jax import lax
from jax.experimental import pallas as pl
from jax.experimental.pallas import tpu as pltpu
from jax.experimental.pallas import tpu_sc as plsc

N = 1024
NI = 4
NT = NI * N
KE = 27
KP = 28
KO = 24
NB = 8
BN = N // NB
NEG = -3.4e38
EPS = 1e-5


def _leaky(x):
    return jnp.where(x >= 0, x, 0.2 * x)


def _dot16(a, b):
    return lax.dot_general(a.astype(jnp.bfloat16), b.astype(jnp.bfloat16),
                           (((1,), (1,)), ((), ())),
                           preferred_element_type=jnp.float32)


def _diag_row(nn2):
    rowi = lax.broadcasted_iota(jnp.int32, (N, N), 0)
    coli = lax.broadcasted_iota(jnp.int32, (N, N), 1)
    d = jnp.where(rowi == coli, jnp.broadcast_to(nn2, (N, N)), 0.0)
    return jnp.sum(d, axis=0, keepdims=True)


RB = 64


def _topk_store(d_scr, idx_ref, k, k_pad, off):
    cols = lax.broadcasted_iota(jnp.int32, (RB, N), 1)
    tpos = lax.broadcasted_iota(jnp.int32, (RB, k_pad), 1)

    def chunk(i, carry):
        d = d_scr[pl.ds(i * RB, RB), :]
        acc = jnp.zeros((RB, k_pad), jnp.int32)
        first = None
        for t in range(k):
            m = jnp.max(d, axis=1, keepdims=True)
            cand = jnp.where(d >= m, cols, jnp.int32(2 * N))
            am = jnp.min(cand, axis=1, keepdims=True)
            acc = jnp.where(tpos == t, am, acc)
            d = jnp.where(cols == am, NEG, d)
            if t == 0:
                first = am
        if k_pad > k:
            acc = jnp.where(tpos >= k, first, acc)
        idx_ref[0, pl.ds(i * RB, RB), :] = acc + off
        return carry

    lax.fori_loop(0, N // RB, chunk, 0)


def _score_blocks(d_scr, load_row_blk, Full, nn_ref_row, nn_query_col):
    for ib in range(N // 128):
        inner = -2.0 * _dot16(load_row_blk(ib), Full)
        d_scr[pl.ds(ib * 128, 128), :] = (
            (-nn_ref_row) - inner) - nn_query_col[ib * 128:(ib + 1) * 128, :]


def _k1_body(p_ref, idx_ref, t0_ref, d_scr):
    q = pl.program_id(0)
    P = p_ref[0]
    nn2 = jnp.sum(P * P, axis=1, keepdims=True)
    nrow = _diag_row(nn2)
    _score_blocks(d_scr, lambda ib: p_ref[0, pl.ds(ib * 128, 128), :],
                  P, nrow, nn2)
    _topk_store(d_scr, idx_ref, KE, KP, q * N)
    t0_ref[0] = jnp.concatenate([P, jnp.zeros((N, 125), jnp.float32)], axis=1)


def _k1(P):
    return pl.pallas_call(
        _k1_body,
        grid=(NI,),
        in_specs=[pl.BlockSpec((1, N, 3), lambda q: (q, 0, 0))],
        out_specs=[
            pl.BlockSpec((1, N, KP), lambda q: (q, 0, 0)),
            pl.BlockSpec((1, N, 128), lambda q: (q, 0, 0)),
        ],
        out_shape=[
            jax.ShapeDtypeStruct((NI, N, KP), jnp.int32),
            jax.ShapeDtypeStruct((NI, N, 128), jnp.float32),
        ],
        scratch_shapes=[pltpu.VMEM((N, N), jnp.float32)],
    )(P)


def _sc_gather(tab, idx_flat, TW):
    M = idx_flat.shape[0]
    NW = 32
    L = M // NW
    CH = 128
    NCH = L // CH
    mesh = plsc.VectorSubcoreMesh(core_axis_name="c", subcore_axis_name="s")

    @functools.partial(
        pl.kernel,
        mesh=mesh,
        out_type=jax.ShapeDtypeStruct((M, TW), jnp.float32),
        scratch_types=[
            pltpu.VMEM((L,), jnp.int32),
            pltpu.VMEM((CH, TW), jnp.float32),
            pltpu.VMEM((CH, TW), jnp.float32),
            pltpu.SemaphoreType.DMA,
            pltpu.SemaphoreType.DMA,
        ],
    )
    def sc_k(tab_hbm, idx_hbm, out_hbm, idx_v, gb0, gb1, sem0, sem1):
        wid = lax.axis_index("s") * 2 + lax.axis_index("c")
        base = wid * L
        pltpu.sync_copy(idx_hbm.at[pl.ds(base, L)], idx_v)
        bufs = (gb0, gb1)
        sems = (sem0, sem1)

        def fire(c):
            pltpu.async_copy(
                tab_hbm.at[idx_v.at[pl.ds(c * CH, CH)]],
                bufs[c % 2], sems[c % 2])

        fire(0)
        for c in range(NCH):
            if c + 1 < NCH:
                fire(c + 1)
            pltpu.make_async_copy(tab_hbm.at[pl.ds(0, CH)], bufs[c % 2],
                                  sems[c % 2]).wait()
            pltpu.sync_copy(bufs[c % 2], out_hbm.at[pl.ds(base + c * CH, CH)])

    return sc_k(tab, idx_flat)


def _acc_write(vals, O, q, ib, mx_ref, mn_ref, s_ref):
    mx = vals[0]
    mn = vals[0]
    s = vals[0]
    qq = vals[0] * vals[0]
    for v in vals[1:]:
        mx = jnp.maximum(mx, v)
        mn = jnp.minimum(mn, v)
        s = s + v
        qq = qq + v * v
    mx_ref[...] = mx
    mn_ref[...] = mn
    part = jnp.concatenate(
        [jnp.sum(s, axis=0, keepdims=True),
         jnp.sum(qq, axis=0, keepdims=True)], axis=0).reshape(1, 2, O)

    @pl.when(jnp.logical_and(q % 2 == 0, ib == 0))
    def _():
        s_ref[...] = jnp.zeros_like(s_ref)

    s_ref[...] += part


def _ke_edge(xg, XC, W, C, TW, kp, kr, O):
    def body(xg_ref, xc_ref, w_ref, mx_ref, mn_ref, s_ref):
        q = pl.program_id(0)
        ib = pl.program_id(1)
        xc = xc_ref[0][:, 0:C]
        feats = [jnp.concatenate([xg_ref[j][:, 0:C] - xc, xc], axis=1)
                 for j in range(kr)]
        out = _dot16(jnp.concatenate(feats, axis=0), w_ref[...])
        vals = [out[j * BN:(j + 1) * BN] for j in range(kr)]
        _acc_write(vals, O, q, ib, mx_ref, mn_ref, s_ref)

    CW = XC.shape[2]
    return pl.pallas_call(
        body,
        grid=(NI, NB),
        in_specs=[
            pl.BlockSpec((kp, BN, TW), lambda q, ib: (0, NB * q + ib, 0)),
            pl.BlockSpec((1, BN, CW), lambda q, ib: (q, ib, 0)),
            pl.BlockSpec((O, 2 * C), lambda q, ib: (0, 0)),
        ],
        out_specs=[
            pl.BlockSpec((BN, O), lambda q, ib: (NB * q + ib, 0)),
            pl.BlockSpec((BN, O), lambda q, ib: (NB * q + ib, 0)),
            pl.BlockSpec((1, 2, O), lambda q, ib: (q // 2, 0, 0)),
        ],
        out_shape=[
            jax.ShapeDtypeStruct((NT, O), jnp.float32),
            jax.ShapeDtypeStruct((NT, O), jnp.float32),
            jax.ShapeDtypeStruct((2, 2, O), jnp.float32),
        ],
    )(xg, XC, W)


def _ke_orient(xg, X3, P, Wo):
    def body(xg_ref, x3_ref, p_ref, w_ref, mx_ref, mn_ref, s_ref):
        q = pl.program_id(0)
        ib = pl.program_id(1)
        ftc = x3_ref[0]
        xtc = p_ref[0]
        feats = []
        for j in range(KO):
            gj = xg_ref[j][:, 0:256]
            zj = xg_ref[j][:, 256:259]
            feats.append(
                jnp.concatenate([gj - ftc, gj, zj - xtc, xtc], axis=1))
        out = _dot16(jnp.concatenate(feats, axis=0), w_ref[...])
        vals = [out[j * BN:(j + 1) * BN] for j in range(KO)]
        _acc_write(vals, 256, q, ib, mx_ref, mn_ref, s_ref)

    return pl.pallas_call(
        body,
        grid=(NI, NB),
        in_specs=[
            pl.BlockSpec((KO, BN, 384), lambda q, ib: (0, NB * q + ib, 0)),
            pl.BlockSpec((1, BN, 256), lambda q, ib: ((q + 2) % 4, ib, 0)),
            pl.BlockSpec((1, BN, 3), lambda q, ib: ((q + 2) % 4, ib, 0)),
            pl.BlockSpec((256, 518), lambda q, ib: (0, 0)),
        ],
        out_specs=[
            pl.BlockSpec((BN, 256), lambda q, ib: (NB * q + ib, 0)),
            pl.BlockSpec((BN, 256), lambda q, ib: (NB * q + ib, 0)),
            pl.BlockSpec((1, 2, 256), lambda q, ib: (q // 2, 0, 0)),
        ],
        out_shape=[
            jax.ShapeDtypeStruct((NT, 256), jnp.float32),
            jax.ShapeDtypeStruct((NT, 256), jnp.float32),
            jax.ShapeDtypeStruct((2, 2, 256), jnp.float32),
        ],
    )(xg, X3, P, Wo)


def _bn_max(mx, mn, s_ref, g_ref, b_ref, kr):
    cnt = float(2 * N * kr)
    mean = s_ref[0, 0:1, :] / cnt
    var = s_ref[0, 1:2, :] / cnt - mean * mean
    den = jnp.sqrt(var + EPS)
    gam = g_ref[...]
    bet = b_ref[...]

    def f(v):
        return _leaky(gam * ((v - mean) / den) + bet)

    return jnp.maximum(f(mx), f(mn))


def _kf2(MX, MN, S, gam, bet, kr, O, PW=None, out3d=False):
    def body(mx_ref, mn_ref, s_ref, g_ref, b_ref, o_ref):
        x = _bn_max(mx_ref[...], mn_ref[...], s_ref, g_ref, b_ref, kr)
        if out3d:
            o_ref[0] = x
        elif PW is not None and PW > O:
            o_ref[...] = jnp.concatenate(
                [x, jnp.zeros((N, PW - O), jnp.float32)], axis=1)
        else:
            o_ref[...] = x

    if out3d:
        out_spec = pl.BlockSpec((1, N, O), lambda q: (q, 0, 0))
        out_shape = jax.ShapeDtypeStruct((NI, N, O), jnp.float32)
    else:
        W2 = PW if PW is not None else O
        out_spec = pl.BlockSpec((N, W2), lambda q: (q, 0))
        out_shape = jax.ShapeDtypeStruct((NT, W2), jnp.float32)
    return pl.pallas_call(
        body,
        grid=(NI,),
        in_specs=[
            pl.BlockSpec((N, O), lambda q: (q, 0)),
            pl.BlockSpec((N, O), lambda q: (q, 0)),
            pl.BlockSpec((1, 2, O), lambda q: (q // 2, 0, 0)),
            pl.BlockSpec((1, O), lambda q: (0, 0)),
            pl.BlockSpec((1, O), lambda q: (0, 0)),
        ],
        out_specs=out_spec,
        out_shape=out_shape,
    )(MX, MN, S, gam, bet)


def _k_orient(x3, P):
    def body(xr_ref, xq_ref, pr_ref, idx_ref, tog_ref, d_scr):
        o = pl.program_id(0)
        F = xr_ref[0]
        Q = xq_ref[0]
        nnF = jnp.sum(F * F, axis=1, keepdims=True)
        nnQ = jnp.sum(Q * Q, axis=1, keepdims=True)
        nrow = _diag_row(nnF)
        _score_blocks(d_scr, lambda ib: xr_ref[0, pl.ds(ib * 128, 128), :],
                      Q, nrow, nnQ)
        _topk_store(d_scr, idx_ref, KO, KO, o * N)
        tog_ref[0] = jnp.concatenate(
            [F, pr_ref[0], jnp.zeros((N, 125), jnp.float32)], axis=1)

    return pl.pallas_call(
        body,
        grid=(NI,),
        in_specs=[
            pl.BlockSpec((1, N, 256), lambda o: (o, 0, 0)),
            pl.BlockSpec((1, N, 256), lambda o: ((o + 2) % 4, 0, 0)),
            pl.BlockSpec((1, N, 3), lambda o: (o, 0, 0)),
        ],
        out_specs=[
            pl.BlockSpec((1, N, KO), lambda o: (o, 0, 0)),
            pl.BlockSpec((1, N, 384), lambda o: (o, 0, 0)),
        ],
        out_shape=[
            jax.ShapeDtypeStruct((NI, N, KO), jnp.int32),
            jax.ShapeDtypeStruct((NI, N, 384), jnp.float32),
        ],
        scratch_shapes=[pltpu.VMEM((N, N), jnp.float32)],
    )(x3, x3, P)


def _k_fo(MX, MN, S, gam, bet):
    def body(mx_ref, mn_ref, s_ref, g_ref, b_ref, l0_ref, idx_ref, d_scr):
        q = pl.program_id(0)
        L = _bn_max(mx_ref[...], mn_ref[...], s_ref, g_ref, b_ref, KO)
        l0_ref[0] = L
        nnL = jnp.sum(L * L, axis=1, keepdims=True)
        nrow = _diag_row(nnL)
        _score_blocks(d_scr, lambda ib: l0_ref[0, pl.ds(ib * 128, 128), :],
                      L, nrow, nnL)
        _topk_store(d_scr, idx_ref, KO, KO, q * N)

    return pl.pallas_call(
        body,
        grid=(NI,),
        in_specs=[
            pl.BlockSpec((N, 256), lambda q: (q, 0)),
            pl.BlockSpec((N, 256), lambda q: (q, 0)),
            pl.BlockSpec((1, 2, 256), lambda q: (q // 2, 0, 0)),
            pl.BlockSpec((1, 256), lambda q: (0, 0)),
            pl.BlockSpec((1, 256), lambda q: (0, 0)),
        ],
        out_specs=[
            pl.BlockSpec((1, N, 256), lambda q: (q, 0, 0)),
            pl.BlockSpec((1, N, KO), lambda q: (q, 0, 0)),
        ],
        out_shape=[
            jax.ShapeDtypeStruct((NI, N, 256), jnp.float32),
            jax.ShapeDtypeStruct((NI, N, KO), jnp.int32),
        ],
        scratch_shapes=[pltpu.VMEM((N, N), jnp.float32)],
    )(MX, MN, S, gam, bet)


def _k_final(MX, MN, S, gam, bet, L0):
    def body(mx_ref, mn_ref, s_ref, g_ref, b_ref, l0_ref, o_ref):
        L1 = _bn_max(mx_ref[...], mn_ref[...], s_ref, g_ref, b_ref, KO)
        cat = jnp.concatenate([l0_ref[0], L1], axis=1)
        cmax = jnp.max(cat, axis=0, keepdims=True)
        cmean = jnp.sum(cat, axis=0, keepdims=True) * (1.0 / N)
        o_ref[...] = jnp.concatenate([cmax, cmean], axis=1).reshape(1, 1, 1024)

    return pl.pallas_call(
        body,
        grid=(NI,),
        in_specs=[
            pl.BlockSpec((N, 256), lambda q: (q, 0)),
            pl.BlockSpec((N, 256), lambda q: (q, 0)),
            pl.BlockSpec((1, 2, 256), lambda q: (q // 2, 0, 0)),
            pl.BlockSpec((1, 256), lambda q: (0, 0)),
            pl.BlockSpec((1, 256), lambda q: (0, 0)),
            pl.BlockSpec((1, N, 256), lambda q: (q, 0, 0)),
        ],
        out_specs=pl.BlockSpec((1, 1, 1024), lambda q: (q, 0, 0)),
        out_shape=jax.ShapeDtypeStruct((NI, 1, 1024), jnp.float32),
    )(MX, MN, S, gam, bet, L0)


def kernel(xyz_s, xyz_t, W0, gamma0, beta0, W1, gamma1, beta1, W2, gamma2,
           beta2, Wo, gammao, betao, We, gammae, betae):
    P = jnp.concatenate([xyz_s, xyz_t], axis=0)
    g0, b0 = gamma0.reshape(1, -1), beta0.reshape(1, -1)
    g1, b1 = gamma1.reshape(1, -1), beta1.reshape(1, -1)
    g2, b2 = gamma2.reshape(1, -1), beta2.reshape(1, -1)
    go, bo = gammao.reshape(1, -1), betao.reshape(1, -1)
    ge, be = gammae.reshape(1, -1), betae.reshape(1, -1)

    idxE, T0g = _k1(P)
    idxE_j = jnp.transpose(idxE[:, :, :KE], (2, 0, 1)).reshape(-1)

    xg0 = _sc_gather(T0g.reshape(NT, 128), idxE_j, 128).reshape(KE, NT, 128)
    MX0, MN0, S0 = _ke_edge(xg0, P, W0, 3, 128, KE, KE, 64)
    T1g = _kf2(MX0, MN0, S0, g0, b0, KE, 64, PW=128)

    xg1 = _sc_gather(T1g, idxE_j, 128).reshape(KE, NT, 128)
    MX1, MN1, S1 = _ke_edge(xg1, T1g.reshape(NI, N, 128), W1, 64, 128,
                            KE, KE, 128)
    T2g = _kf2(MX1, MN1, S1, g1, b1, KE, 128)

    xg2 = _sc_gather(T2g, idxE_j, 128).reshape(KE, NT, 128)
    MX2, MN2, S2 = _ke_edge(xg2, T2g.reshape(NI, N, 128), W2, 128, 128,
                            KE, KE, 256)
    x3 = _kf2(MX2, MN2, S2, g2, b2, KE, 256, out3d=True)

    idxO, Tog = _k_orient(x3, P)
    idxO_j = jnp.transpose(idxO, (2, 0, 1)).reshape(-1)
    xgo = _sc_gather(Tog.reshape(NT, 384), idxO_j, 384).reshape(KO, NT, 384)
    MXo, MNo, So = _ke_orient(xgo, x3, P, Wo)
    L0, idxE2 = _k_fo(MXo, MNo, So, go, bo)

    idxE2_j = jnp.transpose(idxE2, (2, 0, 1)).reshape(-1)
    xge = _sc_gather(L0.reshape(NT, 256), idxE2_j, 256).reshape(KO, NT, 256)
    MXe, MNe, Se = _ke_edge(xge, L0, We, 256, 256, KO, KO, 256)
    OUT = _k_final(MXe, MNe, Se, ge, be, L0)

    OUT = OUT.reshape(NI, 1024)
    xo = OUT[0:2][:, :, None]
    yo = OUT[2:4][:, :, None]
    return xo, yo


# speedup vs baseline: 8.8550x; 1.6679x over previous
import functools

import jax
import jax.numpy as jnp
from jax import lax
from jax.experimental import pallas as pl
from jax.experimental.pallas import tpu as pltpu
from jax.experimental.pallas import tpu_sc as plsc

N = 1024
NI = 4
NT = NI * N
KE = 27
KP = 28
KO = 24
NB = 8
BN = N // NB
NEG = -3.4e38
EPS = 1e-5


def _leaky(x):
    return jnp.where(x >= 0, x, 0.2 * x)


def _dot16(a, b):
    return lax.dot_general(a.astype(jnp.bfloat16), b.astype(jnp.bfloat16),
                           (((1,), (1,)), ((), ())),
                           preferred_element_type=jnp.float32)


def _diag_row(nn2):
    rowi = lax.broadcasted_iota(jnp.int32, (N, N), 0)
    coli = lax.broadcasted_iota(jnp.int32, (N, N), 1)
    d = jnp.where(rowi == coli, jnp.broadcast_to(nn2, (N, N)), 0.0)
    return jnp.sum(d, axis=0, keepdims=True)


RB = 256


def _topk_store(d_scr, idx_ref, k, k_pad, off):
    cols = lax.broadcasted_iota(jnp.int32, (RB, N), 1)
    tpos = lax.broadcasted_iota(jnp.int32, (RB, k_pad), 1)

    def chunk(i, carry):
        d = d_scr[pl.ds(i * RB, RB), :]
        acc = jnp.zeros((RB, k_pad), jnp.int32)
        first = None
        for t in range(k):
            m = jnp.max(d, axis=1, keepdims=True)
            cand = jnp.where(d >= m, cols, jnp.int32(2 * N))
            am = jnp.min(cand, axis=1, keepdims=True)
            acc = jnp.where(tpos == t, am, acc)
            d = jnp.where(cols == am, NEG, d)
            if t == 0:
                first = am
        if k_pad > k:
            acc = jnp.where(tpos >= k, first, acc)
        idx_ref[0, pl.ds(i * RB, RB), :] = acc + off
        return carry

    lax.fori_loop(0, N // RB, chunk, 0)


def _score_blocks(d_scr, load_row_blk, Full, nn_ref_row, nn_query_col):
    for ib in range(N // 128):
        inner = -2.0 * _dot16(load_row_blk(ib), Full)
        d_scr[pl.ds(ib * 128, 128), :] = (
            (-nn_ref_row) - inner) - nn_query_col[ib * 128:(ib + 1) * 128, :]


def _k1_body(p_ref, idx_ref, t0_ref, d_scr):
    q = pl.program_id(0)
    P = p_ref[0]
    nn2 = jnp.sum(P * P, axis=1, keepdims=True)
    nrow = _diag_row(nn2)
    _score_blocks(d_scr, lambda ib: p_ref[0, pl.ds(ib * 128, 128), :],
                  P, nrow, nn2)
    _topk_store(d_scr, idx_ref, KE, KP, q * N)
    t0_ref[0] = jnp.concatenate([P, jnp.zeros((N, 125), jnp.float32)], axis=1)


def _k1(P):
    return pl.pallas_call(
        _k1_body,
        grid=(NI,),
        in_specs=[pl.BlockSpec((1, N, 3), lambda q: (q, 0, 0))],
        out_specs=[
            pl.BlockSpec((1, N, KP), lambda q: (q, 0, 0)),
            pl.BlockSpec((1, N, 128), lambda q: (q, 0, 0)),
        ],
        out_shape=[
            jax.ShapeDtypeStruct((NI, N, KP), jnp.int32),
            jax.ShapeDtypeStruct((NI, N, 128), jnp.float32),
        ],
        scratch_shapes=[pltpu.VMEM((N, N), jnp.float32)],
    )(P)


def _sc_gather(tab, idx_flat, TW):
    M = idx_flat.shape[0]
    NW = 32
    L = M // NW
    CH = 128
    NCH = L // CH
    mesh = plsc.VectorSubcoreMesh(core_axis_name="c", subcore_axis_name="s")

    @functools.partial(
        pl.kernel,
        mesh=mesh,
        out_type=jax.ShapeDtypeStruct((M, TW), jnp.float32),
        scratch_types=[
            pltpu.VMEM((L,), jnp.int32),
            pltpu.VMEM((CH, TW), jnp.float32),
            pltpu.VMEM((CH, TW), jnp.float32),
            pltpu.SemaphoreType.DMA,
            pltpu.SemaphoreType.DMA,
        ],
    )
    def sc_k(tab_hbm, idx_hbm, out_hbm, idx_v, gb0, gb1, sem0, sem1):
        wid = lax.axis_index("s") * 2 + lax.axis_index("c")
        base = wid * L
        pltpu.sync_copy(idx_hbm.at[pl.ds(base, L)], idx_v)
        bufs = (gb0, gb1)
        sems = (sem0, sem1)

        def fire(c):
            pltpu.async_copy(
                tab_hbm.at[idx_v.at[pl.ds(c * CH, CH)]],
                bufs[c % 2], sems[c % 2])

        fire(0)
        for c in range(NCH):
            if c + 1 < NCH:
                fire(c + 1)
            pltpu.make_async_copy(tab_hbm.at[pl.ds(0, CH)], bufs[c % 2],
                                  sems[c % 2]).wait()
            pltpu.sync_copy(bufs[c % 2], out_hbm.at[pl.ds(base + c * CH, CH)])

    return sc_k(tab, idx_flat)


def _acc_write(vals, O, q, ib, mx_ref, mn_ref, s_ref):
    mx = vals[0]
    mn = vals[0]
    s = vals[0]
    qq = vals[0] * vals[0]
    for v in vals[1:]:
        mx = jnp.maximum(mx, v)
        mn = jnp.minimum(mn, v)
        s = s + v
        qq = qq + v * v
    mx_ref[...] = mx
    mn_ref[...] = mn
    part = jnp.concatenate(
        [jnp.sum(s, axis=0, keepdims=True),
         jnp.sum(qq, axis=0, keepdims=True)], axis=0).reshape(1, 2, O)

    @pl.when(jnp.logical_and(q % 2 == 0, ib == 0))
    def _():
        s_ref[...] = jnp.zeros_like(s_ref)

    s_ref[...] += part


def _ke_edge(xg, XC, W, C, TW, kp, kr, O):
    def body(xg_ref, xc_ref, w_ref, mx_ref, mn_ref, s_ref):
        q = pl.program_id(0)
        ib = pl.program_id(1)
        xc = xc_ref[0][:, 0:C]
        feats = [jnp.concatenate([xg_ref[j][:, 0:C] - xc, xc], axis=1)
                 for j in range(kr)]
        out = _dot16(jnp.concatenate(feats, axis=0), w_ref[...])
        vals = [out[j * BN:(j + 1) * BN] for j in range(kr)]
        _acc_write(vals, O, q, ib, mx_ref, mn_ref, s_ref)

    CW = XC.shape[2]
    return pl.pallas_call(
        body,
        grid=(NI, NB),
        in_specs=[
            pl.BlockSpec((kp, BN, TW), lambda q, ib: (0, NB * q + ib, 0)),
            pl.BlockSpec((1, BN, CW), lambda q, ib: (q, ib, 0)),
            pl.BlockSpec((O, 2 * C), lambda q, ib: (0, 0)),
        ],
        out_specs=[
            pl.BlockSpec((BN, O), lambda q, ib: (NB * q + ib, 0)),
            pl.BlockSpec((BN, O), lambda q, ib: (NB * q + ib, 0)),
            pl.BlockSpec((1, 2, O), lambda q, ib: (q // 2, 0, 0)),
        ],
        out_shape=[
            jax.ShapeDtypeStruct((NT, O), jnp.float32),
            jax.ShapeDtypeStruct((NT, O), jnp.float32),
            jax.ShapeDtypeStruct((2, 2, O), jnp.float32),
        ],
    )(xg, XC, W)


def _ke_orient(xg, X3, P, Wo):
    def body(xg_ref, x3_ref, p_ref, w_ref, mx_ref, mn_ref, s_ref):
        q = pl.program_id(0)
        ib = pl.program_id(1)
        ftc = x3_ref[0]
        xtc = p_ref[0]
        feats = []
        for j in range(KO):
            gj = xg_ref[j][:, 0:256]
            zj = xg_ref[j][:, 256:259]
            feats.append(
                jnp.concatenate([gj - ftc, gj, zj - xtc, xtc], axis=1))
        out = _dot16(jnp.concatenate(feats, axis=0), w_ref[...])
        vals = [out[j * BN:(j + 1) * BN] for j in range(KO)]
        _acc_write(vals, 256, q, ib, mx_ref, mn_ref, s_ref)

    return pl.pallas_call(
        body,
        grid=(NI, NB),
        in_specs=[
            pl.BlockSpec((KO, BN, 384), lambda q, ib: (0, NB * q + ib, 0)),
            pl.BlockSpec((1, BN, 256), lambda q, ib: ((q + 2) % 4, ib, 0)),
            pl.BlockSpec((1, BN, 3), lambda q, ib: ((q + 2) % 4, ib, 0)),
            pl.BlockSpec((256, 518), lambda q, ib: (0, 0)),
        ],
        out_specs=[
            pl.BlockSpec((BN, 256), lambda q, ib: (NB * q + ib, 0)),
            pl.BlockSpec((BN, 256), lambda q, ib: (NB * q + ib, 0)),
            pl.BlockSpec((1, 2, 256), lambda q, ib: (q // 2, 0, 0)),
        ],
        out_shape=[
            jax.ShapeDtypeStruct((NT, 256), jnp.float32),
            jax.ShapeDtypeStruct((NT, 256), jnp.float32),
            jax.ShapeDtypeStruct((2, 2, 256), jnp.float32),
        ],
    )(xg, X3, P, Wo)


def _bn_max(mx, mn, s_ref, g_ref, b_ref, kr):
    cnt = float(2 * N * kr)
    mean = s_ref[0, 0:1, :] / cnt
    var = s_ref[0, 1:2, :] / cnt - mean * mean
    den = jnp.sqrt(var + EPS)
    gam = g_ref[...]
    bet = b_ref[...]

    def f(v):
        return _leaky(gam * ((v - mean) / den) + bet)

    return jnp.maximum(f(mx), f(mn))


def _kf2(MX, MN, S, gam, bet, kr, O, PW=None, out3d=False):
    def body(mx_ref, mn_ref, s_ref, g_ref, b_ref, o_ref):
        x = _bn_max(mx_ref[...], mn_ref[...], s_ref, g_ref, b_ref, kr)
        if out3d:
            o_ref[0] = x
        elif PW is not None and PW > O:
            o_ref[...] = jnp.concatenate(
                [x, jnp.zeros((N, PW - O), jnp.float32)], axis=1)
        else:
            o_ref[...] = x

    if out3d:
        out_spec = pl.BlockSpec((1, N, O), lambda q: (q, 0, 0))
        out_shape = jax.ShapeDtypeStruct((NI, N, O), jnp.float32)
    else:
        W2 = PW if PW is not None else O
        out_spec = pl.BlockSpec((N, W2), lambda q: (q, 0))
        out_shape = jax.ShapeDtypeStruct((NT, W2), jnp.float32)
    return pl.pallas_call(
        body,
        grid=(NI,),
        in_specs=[
            pl.BlockSpec((N, O), lambda q: (q, 0)),
            pl.BlockSpec((N, O), lambda q: (q, 0)),
            pl.BlockSpec((1, 2, O), lambda q: (q // 2, 0, 0)),
            pl.BlockSpec((1, O), lambda q: (0, 0)),
            pl.BlockSpec((1, O), lambda q: (0, 0)),
        ],
        out_specs=out_spec,
        out_shape=out_shape,
    )(MX, MN, S, gam, bet)


def _k_orient(x3, P):
    def body(xr_ref, xq_ref, pr_ref, idx_ref, tog_ref, d_scr):
        o = pl.program_id(0)
        F = xr_ref[0]
        Q = xq_ref[0]
        nnF = jnp.sum(F * F, axis=1, keepdims=True)
        nnQ = jnp.sum(Q * Q, axis=1, keepdims=True)
        nrow = _diag_row(nnF)
        _score_blocks(d_scr, lambda ib: xr_ref[0, pl.ds(ib * 128, 128), :],
                      Q, nrow, nnQ)
        _topk_store(d_scr, idx_ref, KO, KO, o * N)
        tog_ref[0] = jnp.concatenate(
            [F, pr_ref[0], jnp.zeros((N, 125), jnp.float32)], axis=1)

    return pl.pallas_call(
        body,
        grid=(NI,),
        in_specs=[
            pl.BlockSpec((1, N, 256), lambda o: (o, 0, 0)),
            pl.BlockSpec((1, N, 256), lambda o: ((o + 2) % 4, 0, 0)),
            pl.BlockSpec((1, N, 3), lambda o: (o, 0, 0)),
        ],
        out_specs=[
            pl.BlockSpec((1, N, KO), lambda o: (o, 0, 0)),
            pl.BlockSpec((1, N, 384), lambda o: (o, 0, 0)),
        ],
        out_shape=[
            jax.ShapeDtypeStruct((NI, N, KO), jnp.int32),
            jax.ShapeDtypeStruct((NI, N, 384), jnp.float32),
        ],
        scratch_shapes=[pltpu.VMEM((N, N), jnp.float32)],
    )(x3, x3, P)


def _k_fo(MX, MN, S, gam, bet):
    def body(mx_ref, mn_ref, s_ref, g_ref, b_ref, l0_ref, idx_ref, d_scr):
        q = pl.program_id(0)
        L = _bn_max(mx_ref[...], mn_ref[...], s_ref, g_ref, b_ref, KO)
        l0_ref[0] = L
        nnL = jnp.sum(L * L, axis=1, keepdims=True)
        nrow = _diag_row(nnL)
        _score_blocks(d_scr, lambda ib: l0_ref[0, pl.ds(ib * 128, 128), :],
                      L, nrow, nnL)
        _topk_store(d_scr, idx_ref, KO, KO, q * N)

    return pl.pallas_call(
        body,
        grid=(NI,),
        in_specs=[
            pl.BlockSpec((N, 256), lambda q: (q, 0)),
            pl.BlockSpec((N, 256), lambda q: (q, 0)),
            pl.BlockSpec((1, 2, 256), lambda q: (q // 2, 0, 0)),
            pl.BlockSpec((1, 256), lambda q: (0, 0)),
            pl.BlockSpec((1, 256), lambda q: (0, 0)),
        ],
        out_specs=[
            pl.BlockSpec((1, N, 256), lambda q: (q, 0, 0)),
            pl.BlockSpec((1, N, KO), lambda q: (q, 0, 0)),
        ],
        out_shape=[
            jax.ShapeDtypeStruct((NI, N, 256), jnp.float32),
            jax.ShapeDtypeStruct((NI, N, KO), jnp.int32),
        ],
        scratch_shapes=[pltpu.VMEM((N, N), jnp.float32)],
    )(MX, MN, S, gam, bet)


def _k_final(MX, MN, S, gam, bet, L0):
    def body(mx_ref, mn_ref, s_ref, g_ref, b_ref, l0_ref, o_ref):
        L1 = _bn_max(mx_ref[...], mn_ref[...], s_ref, g_ref, b_ref, KO)
        cat = jnp.concatenate([l0_ref[0], L1], axis=1)
        cmax = jnp.max(cat, axis=0, keepdims=True)
        cmean = jnp.sum(cat, axis=0, keepdims=True) * (1.0 / N)
        o_ref[...] = jnp.concatenate([cmax, cmean], axis=1).reshape(1, 1, 1024)

    return pl.pallas_call(
        body,
        grid=(NI,),
        in_specs=[
            pl.BlockSpec((N, 256), lambda q: (q, 0)),
            pl.BlockSpec((N, 256), lambda q: (q, 0)),
            pl.BlockSpec((1, 2, 256), lambda q: (q // 2, 0, 0)),
            pl.BlockSpec((1, 256), lambda q: (0, 0)),
            pl.BlockSpec((1, 256), lambda q: (0, 0)),
            pl.BlockSpec((1, N, 256), lambda q: (q, 0, 0)),
        ],
        out_specs=pl.BlockSpec((1, 1, 1024), lambda q: (q, 0, 0)),
        out_shape=jax.ShapeDtypeStruct((NI, 1, 1024), jnp.float32),
    )(MX, MN, S, gam, bet, L0)


def kernel(xyz_s, xyz_t, W0, gamma0, beta0, W1, gamma1, beta1, W2, gamma2,
           beta2, Wo, gammao, betao, We, gammae, betae):
    P = jnp.concatenate([xyz_s, xyz_t], axis=0)
    g0, b0 = gamma0.reshape(1, -1), beta0.reshape(1, -1)
    g1, b1 = gamma1.reshape(1, -1), beta1.reshape(1, -1)
    g2, b2 = gamma2.reshape(1, -1), beta2.reshape(1, -1)
    go, bo = gammao.reshape(1, -1), betao.reshape(1, -1)
    ge, be = gammae.reshape(1, -1), betae.reshape(1, -1)

    idxE, T0g = _k1(P)
    idxE_j = jnp.transpose(idxE[:, :, :KE], (2, 0, 1)).reshape(-1)

    xg0 = _sc_gather(T0g.reshape(NT, 128), idxE_j, 128).reshape(KE, NT, 128)
    MX0, MN0, S0 = _ke_edge(xg0, P, W0, 3, 128, KE, KE, 64)
    T1g = _kf2(MX0, MN0, S0, g0, b0, KE, 64, PW=128)

    xg1 = _sc_gather(T1g, idxE_j, 128).reshape(KE, NT, 128)
    MX1, MN1, S1 = _ke_edge(xg1, T1g.reshape(NI, N, 128), W1, 64, 128,
                            KE, KE, 128)
    T2g = _kf2(MX1, MN1, S1, g1, b1, KE, 128)

    xg2 = _sc_gather(T2g, idxE_j, 128).reshape(KE, NT, 128)
    MX2, MN2, S2 = _ke_edge(xg2, T2g.reshape(NI, N, 128), W2, 128, 128,
                            KE, KE, 256)
    x3 = _kf2(MX2, MN2, S2, g2, b2, KE, 256, out3d=True)

    idxO, Tog = _k_orient(x3, P)
    idxO_j = jnp.transpose(idxO, (2, 0, 1)).reshape(-1)
    xgo = _sc_gather(Tog.reshape(NT, 384), idxO_j, 384).reshape(KO, NT, 384)
    MXo, MNo, So = _ke_orient(xgo, x3, P, Wo)
    L0, idxE2 = _k_fo(MXo, MNo, So, go, bo)

    idxE2_j = jnp.transpose(idxE2, (2, 0, 1)).reshape(-1)
    xge = _sc_gather(L0.reshape(NT, 256), idxE2_j, 256).reshape(KO, NT, 256)
    MXe, MNe, Se = _ke_edge(xge, L0, We, 256, 256, KO, KO, 256)
    OUT = _k_final(MXe, MNe, Se, ge, be, L0)

    OUT = OUT.reshape(NI, 1024)
    xo = OUT[0:2][:, :, None]
    yo = OUT[2:4][:, :, None]
    return xo, yo


# speedup vs baseline: 9.4246x; 1.0643x over previous
import functools

import jax
import jax.numpy as jnp
from jax import lax
from jax.experimental import pallas as pl
from jax.experimental.pallas import tpu as pltpu
from jax.experimental.pallas import tpu_sc as plsc

N = 1024
NI = 4
NT = NI * N
KE = 27
KP = 28
KO = 24
NB = 8
BN = N // NB
NEG = -3.4e38
EPS = 1e-5


def _leaky(x):
    return jnp.where(x >= 0, x, 0.2 * x)


def _dot16(a, b):
    return lax.dot_general(a.astype(jnp.bfloat16), b.astype(jnp.bfloat16),
                           (((1,), (1,)), ((), ())),
                           preferred_element_type=jnp.float32)


def _diag_row(nn2):
    rowi = lax.broadcasted_iota(jnp.int32, (N, N), 0)
    coli = lax.broadcasted_iota(jnp.int32, (N, N), 1)
    d = jnp.where(rowi == coli, jnp.broadcast_to(nn2, (N, N)), 0.0)
    return jnp.sum(d, axis=0, keepdims=True)


RB = 512


def _topk_store(d_scr, idx_ref, k, k_pad, off):
    cols = lax.broadcasted_iota(jnp.int32, (RB, N), 1)
    tpos = lax.broadcasted_iota(jnp.int32, (RB, k_pad), 1)

    def chunk(i, carry):
        d = d_scr[pl.ds(i * RB, RB), :]
        acc = jnp.zeros((RB, k_pad), jnp.int32)
        first = None
        for t in range(k):
            m = jnp.max(d, axis=1, keepdims=True)
            cand = jnp.where(d >= m, cols, jnp.int32(2 * N))
            am = jnp.min(cand, axis=1, keepdims=True)
            acc = jnp.where(tpos == t, am, acc)
            d = jnp.where(cols == am, NEG, d)
            if t == 0:
                first = am
        if k_pad > k:
            acc = jnp.where(tpos >= k, first, acc)
        idx_ref[0, pl.ds(i * RB, RB), :] = acc + off
        return carry

    lax.fori_loop(0, N // RB, chunk, 0)


def _score_blocks(d_scr, load_row_blk, Full, nn_ref_row, nn_query_col):
    for ib in range(N // 128):
        inner = -2.0 * _dot16(load_row_blk(ib), Full)
        d_scr[pl.ds(ib * 128, 128), :] = (
            (-nn_ref_row) - inner) - nn_query_col[ib * 128:(ib + 1) * 128, :]


def _k1_body(p_ref, idx_ref, t0_ref, d_scr):
    q = pl.program_id(0)
    P = p_ref[0]
    nn2 = jnp.sum(P * P, axis=1, keepdims=True)
    nrow = _diag_row(nn2)
    _score_blocks(d_scr, lambda ib: p_ref[0, pl.ds(ib * 128, 128), :],
                  P, nrow, nn2)
    _topk_store(d_scr, idx_ref, KE, KP, q * N)
    t0_ref[0] = jnp.concatenate([P, jnp.zeros((N, 125), jnp.float32)], axis=1)


def _k1(P):
    return pl.pallas_call(
        _k1_body,
        grid=(NI,),
        in_specs=[pl.BlockSpec((1, N, 3), lambda q: (q, 0, 0))],
        out_specs=[
            pl.BlockSpec((1, N, KP), lambda q: (q, 0, 0)),
            pl.BlockSpec((1, N, 128), lambda q: (q, 0, 0)),
        ],
        out_shape=[
            jax.ShapeDtypeStruct((NI, N, KP), jnp.int32),
            jax.ShapeDtypeStruct((NI, N, 128), jnp.float32),
        ],
        scratch_shapes=[pltpu.VMEM((N, N), jnp.float32)],
    )(P)


def _sc_gather(tab, idx_flat, TW):
    M = idx_flat.shape[0]
    NW = 32
    L = M // NW
    CH = 128
    NCH = L // CH
    mesh = plsc.VectorSubcoreMesh(core_axis_name="c", subcore_axis_name="s")

    @functools.partial(
        pl.kernel,
        mesh=mesh,
        out_type=jax.ShapeDtypeStruct((M, TW), jnp.float32),
        scratch_types=[
            pltpu.VMEM((L,), jnp.int32),
            pltpu.VMEM((CH, TW), jnp.float32),
            pltpu.VMEM((CH, TW), jnp.float32),
            pltpu.SemaphoreType.DMA,
            pltpu.SemaphoreType.DMA,
        ],
    )
    def sc_k(tab_hbm, idx_hbm, out_hbm, idx_v, gb0, gb1, sem0, sem1):
        wid = lax.axis_index("s") * 2 + lax.axis_index("c")
        base = wid * L
        pltpu.sync_copy(idx_hbm.at[pl.ds(base, L)], idx_v)
        bufs = (gb0, gb1)
        sems = (sem0, sem1)

        def fire(c):
            pltpu.async_copy(
                tab_hbm.at[idx_v.at[pl.ds(c * CH, CH)]],
                bufs[c % 2], sems[c % 2])

        fire(0)
        for c in range(NCH):
            if c + 1 < NCH:
                fire(c + 1)
            pltpu.make_async_copy(tab_hbm.at[pl.ds(0, CH)], bufs[c % 2],
                                  sems[c % 2]).wait()
            pltpu.sync_copy(bufs[c % 2], out_hbm.at[pl.ds(base + c * CH, CH)])

    return sc_k(tab, idx_flat)


def _acc_write(vals, O, q, ib, mx_ref, mn_ref, s_ref):
    mx = vals[0]
    mn = vals[0]
    s = vals[0]
    qq = vals[0] * vals[0]
    for v in vals[1:]:
        mx = jnp.maximum(mx, v)
        mn = jnp.minimum(mn, v)
        s = s + v
        qq = qq + v * v
    mx_ref[...] = mx
    mn_ref[...] = mn
    part = jnp.concatenate(
        [jnp.sum(s, axis=0, keepdims=True),
         jnp.sum(qq, axis=0, keepdims=True)], axis=0).reshape(1, 2, O)

    @pl.when(jnp.logical_and(q % 2 == 0, ib == 0))
    def _():
        s_ref[...] = jnp.zeros_like(s_ref)

    s_ref[...] += part


def _ke_edge(xg, XC, W, C, TW, kp, kr, O):
    def body(xg_ref, xc_ref, w_ref, mx_ref, mn_ref, s_ref):
        q = pl.program_id(0)
        ib = pl.program_id(1)
        xc = xc_ref[0][:, 0:C]
        feats = [jnp.concatenate([xg_ref[j][:, 0:C] - xc, xc], axis=1)
                 for j in range(kr)]
        out = _dot16(jnp.concatenate(feats, axis=0), w_ref[...])
        vals = [out[j * BN:(j + 1) * BN] for j in range(kr)]
        _acc_write(vals, O, q, ib, mx_ref, mn_ref, s_ref)

    CW = XC.shape[2]
    return pl.pallas_call(
        body,
        grid=(NI, NB),
        in_specs=[
            pl.BlockSpec((kp, BN, TW), lambda q, ib: (0, NB * q + ib, 0)),
            pl.BlockSpec((1, BN, CW), lambda q, ib: (q, ib, 0)),
            pl.BlockSpec((O, 2 * C), lambda q, ib: (0, 0)),
        ],
        out_specs=[
            pl.BlockSpec((BN, O), lambda q, ib: (NB * q + ib, 0)),
            pl.BlockSpec((BN, O), lambda q, ib: (NB * q + ib, 0)),
            pl.BlockSpec((1, 2, O), lambda q, ib: (q // 2, 0, 0)),
        ],
        out_shape=[
            jax.ShapeDtypeStruct((NT, O), jnp.float32),
            jax.ShapeDtypeStruct((NT, O), jnp.float32),
            jax.ShapeDtypeStruct((2, 2, O), jnp.float32),
        ],
    )(xg, XC, W)


def _ke_orient(xg, X3, P, Wo):
    def body(xg_ref, x3_ref, p_ref, w_ref, mx_ref, mn_ref, s_ref):
        q = pl.program_id(0)
        ib = pl.program_id(1)
        ftc = x3_ref[0]
        xtc = p_ref[0]
        feats = []
        for j in range(KO):
            gj = xg_ref[j][:, 0:256]
            zj = xg_ref[j][:, 256:259]
            feats.append(
                jnp.concatenate([gj - ftc, gj, zj - xtc, xtc], axis=1))
        out = _dot16(jnp.concatenate(feats, axis=0), w_ref[...])
        vals = [out[j * BN:(j + 1) * BN] for j in range(KO)]
        _acc_write(vals, 256, q, ib, mx_ref, mn_ref, s_ref)

    return pl.pallas_call(
        body,
        grid=(NI, NB),
        in_specs=[
            pl.BlockSpec((KO, BN, 384), lambda q, ib: (0, NB * q + ib, 0)),
            pl.BlockSpec((1, BN, 256), lambda q, ib: ((q + 2) % 4, ib, 0)),
            pl.BlockSpec((1, BN, 3), lambda q, ib: ((q + 2) % 4, ib, 0)),
            pl.BlockSpec((256, 518), lambda q, ib: (0, 0)),
        ],
        out_specs=[
            pl.BlockSpec((BN, 256), lambda q, ib: (NB * q + ib, 0)),
            pl.BlockSpec((BN, 256), lambda q, ib: (NB * q + ib, 0)),
            pl.BlockSpec((1, 2, 256), lambda q, ib: (q // 2, 0, 0)),
        ],
        out_shape=[
            jax.ShapeDtypeStruct((NT, 256), jnp.float32),
            jax.ShapeDtypeStruct((NT, 256), jnp.float32),
            jax.ShapeDtypeStruct((2, 2, 256), jnp.float32),
        ],
    )(xg, X3, P, Wo)


def _bn_max(mx, mn, s_ref, g_ref, b_ref, kr):
    cnt = float(2 * N * kr)
    mean = s_ref[0, 0:1, :] / cnt
    var = s_ref[0, 1:2, :] / cnt - mean * mean
    den = jnp.sqrt(var + EPS)
    gam = g_ref[...]
    bet = b_ref[...]

    def f(v):
        return _leaky(gam * ((v - mean) / den) + bet)

    return jnp.maximum(f(mx), f(mn))


def _kf2(MX, MN, S, gam, bet, kr, O, PW=None, out3d=False):
    def body(mx_ref, mn_ref, s_ref, g_ref, b_ref, o_ref):
        x = _bn_max(mx_ref[...], mn_ref[...], s_ref, g_ref, b_ref, kr)
        if out3d:
            o_ref[0] = x
        elif PW is not None and PW > O:
            o_ref[...] = jnp.concatenate(
                [x, jnp.zeros((N, PW - O), jnp.float32)], axis=1)
        else:
            o_ref[...] = x

    if out3d:
        out_spec = pl.BlockSpec((1, N, O), lambda q: (q, 0, 0))
        out_shape = jax.ShapeDtypeStruct((NI, N, O), jnp.float32)
    else:
        W2 = PW if PW is not None else O
        out_spec = pl.BlockSpec((N, W2), lambda q: (q, 0))
        out_shape = jax.ShapeDtypeStruct((NT, W2), jnp.float32)
    return pl.pallas_call(
        body,
        grid=(NI,),
        in_specs=[
            pl.BlockSpec((N, O), lambda q: (q, 0)),
            pl.BlockSpec((N, O), lambda q: (q, 0)),
            pl.BlockSpec((1, 2, O), lambda q: (q // 2, 0, 0)),
            pl.BlockSpec((1, O), lambda q: (0, 0)),
            pl.BlockSpec((1, O), lambda q: (0, 0)),
        ],
        out_specs=out_spec,
        out_shape=out_shape,
    )(MX, MN, S, gam, bet)


def _k_orient(x3, P):
    def body(xr_ref, xq_ref, pr_ref, idx_ref, tog_ref, d_scr):
        o = pl.program_id(0)
        F = xr_ref[0]
        Q = xq_ref[0]
        nnF = jnp.sum(F * F, axis=1, keepdims=True)
        nnQ = jnp.sum(Q * Q, axis=1, keepdims=True)
        nrow = _diag_row(nnF)
        _score_blocks(d_scr, lambda ib: xr_ref[0, pl.ds(ib * 128, 128), :],
                      Q, nrow, nnQ)
        _topk_store(d_scr, idx_ref, KO, KO, o * N)
        tog_ref[0] = jnp.concatenate(
            [F, pr_ref[0], jnp.zeros((N, 125), jnp.float32)], axis=1)

    return pl.pallas_call(
        body,
        grid=(NI,),
        in_specs=[
            pl.BlockSpec((1, N, 256), lambda o: (o, 0, 0)),
            pl.BlockSpec((1, N, 256), lambda o: ((o + 2) % 4, 0, 0)),
            pl.BlockSpec((1, N, 3), lambda o: (o, 0, 0)),
        ],
        out_specs=[
            pl.BlockSpec((1, N, KO), lambda o: (o, 0, 0)),
            pl.BlockSpec((1, N, 384), lambda o: (o, 0, 0)),
        ],
        out_shape=[
            jax.ShapeDtypeStruct((NI, N, KO), jnp.int32),
            jax.ShapeDtypeStruct((NI, N, 384), jnp.float32),
        ],
        scratch_shapes=[pltpu.VMEM((N, N), jnp.float32)],
    )(x3, x3, P)


def _k_fo(MX, MN, S, gam, bet):
    def body(mx_ref, mn_ref, s_ref, g_ref, b_ref, l0_ref, idx_ref, d_scr):
        q = pl.program_id(0)
        L = _bn_max(mx_ref[...], mn_ref[...], s_ref, g_ref, b_ref, KO)
        l0_ref[0] = L
        nnL = jnp.sum(L * L, axis=1, keepdims=True)
        nrow = _diag_row(nnL)
        _score_blocks(d_scr, lambda ib: l0_ref[0, pl.ds(ib * 128, 128), :],
                      L, nrow, nnL)
        _topk_store(d_scr, idx_ref, KO, KO, q * N)

    return pl.pallas_call(
        body,
        grid=(NI,),
        in_specs=[
            pl.BlockSpec((N, 256), lambda q: (q, 0)),
            pl.BlockSpec((N, 256), lambda q: (q, 0)),
            pl.BlockSpec((1, 2, 256), lambda q: (q // 2, 0, 0)),
            pl.BlockSpec((1, 256), lambda q: (0, 0)),
            pl.BlockSpec((1, 256), lambda q: (0, 0)),
        ],
        out_specs=[
            pl.BlockSpec((1, N, 256), lambda q: (q, 0, 0)),
            pl.BlockSpec((1, N, KO), lambda q: (q, 0, 0)),
        ],
        out_shape=[
            jax.ShapeDtypeStruct((NI, N, 256), jnp.float32),
            jax.ShapeDtypeStruct((NI, N, KO), jnp.int32),
        ],
        scratch_shapes=[pltpu.VMEM((N, N), jnp.float32)],
    )(MX, MN, S, gam, bet)


def _k_final(MX, MN, S, gam, bet, L0):
    def body(mx_ref, mn_ref, s_ref, g_ref, b_ref, l0_ref, o_ref):
        L1 = _bn_max(mx_ref[...], mn_ref[...], s_ref, g_ref, b_ref, KO)
        cat = jnp.concatenate([l0_ref[0], L1], axis=1)
        cmax = jnp.max(cat, axis=0, keepdims=True)
        cmean = jnp.sum(cat, axis=0, keepdims=True) * (1.0 / N)
        o_ref[...] = jnp.concatenate([cmax, cmean], axis=1).reshape(1, 1, 1024)

    return pl.pallas_call(
        body,
        grid=(NI,),
        in_specs=[
            pl.BlockSpec((N, 256), lambda q: (q, 0)),
            pl.BlockSpec((N, 256), lambda q: (q, 0)),
            pl.BlockSpec((1, 2, 256), lambda q: (q // 2, 0, 0)),
            pl.BlockSpec((1, 256), lambda q: (0, 0)),
            pl.BlockSpec((1, 256), lambda q: (0, 0)),
            pl.BlockSpec((1, N, 256), lambda q: (q, 0, 0)),
        ],
        out_specs=pl.BlockSpec((1, 1, 1024), lambda q: (q, 0, 0)),
        out_shape=jax.ShapeDtypeStruct((NI, 1, 1024), jnp.float32),
    )(MX, MN, S, gam, bet, L0)


def kernel(xyz_s, xyz_t, W0, gamma0, beta0, W1, gamma1, beta1, W2, gamma2,
           beta2, Wo, gammao, betao, We, gammae, betae):
    P = jnp.concatenate([xyz_s, xyz_t], axis=0)
    g0, b0 = gamma0.reshape(1, -1), beta0.reshape(1, -1)
    g1, b1 = gamma1.reshape(1, -1), beta1.reshape(1, -1)
    g2, b2 = gamma2.reshape(1, -1), beta2.reshape(1, -1)
    go, bo = gammao.reshape(1, -1), betao.reshape(1, -1)
    ge, be = gammae.reshape(1, -1), betae.reshape(1, -1)

    idxE, T0g = _k1(P)
    idxE_j = jnp.transpose(idxE[:, :, :KE], (2, 0, 1)).reshape(-1)

    xg0 = _sc_gather(T0g.reshape(NT, 128), idxE_j, 128).reshape(KE, NT, 128)
    MX0, MN0, S0 = _ke_edge(xg0, P, W0, 3, 128, KE, KE, 64)
    T1g = _kf2(MX0, MN0, S0, g0, b0, KE, 64, PW=128)

    xg1 = _sc_gather(T1g, idxE_j, 128).reshape(KE, NT, 128)
    MX1, MN1, S1 = _ke_edge(xg1, T1g.reshape(NI, N, 128), W1, 64, 128,
                            KE, KE, 128)
    T2g = _kf2(MX1, MN1, S1, g1, b1, KE, 128)

    xg2 = _sc_gather(T2g, idxE_j, 128).reshape(KE, NT, 128)
    MX2, MN2, S2 = _ke_edge(xg2, T2g.reshape(NI, N, 128), W2, 128, 128,
                            KE, KE, 256)
    x3 = _kf2(MX2, MN2, S2, g2, b2, KE, 256, out3d=True)

    idxO, Tog = _k_orient(x3, P)
    idxO_j = jnp.transpose(idxO, (2, 0, 1)).reshape(-1)
    xgo = _sc_gather(Tog.reshape(NT, 384), idxO_j, 384).reshape(KO, NT, 384)
    MXo, MNo, So = _ke_orient(xgo, x3, P, Wo)
    L0, idxE2 = _k_fo(MXo, MNo, So, go, bo)

    idxE2_j = jnp.transpose(idxE2, (2, 0, 1)).reshape(-1)
    xge = _sc_gather(L0.reshape(NT, 256), idxE2_j, 256).reshape(KO, NT, 256)
    MXe, MNe, Se = _ke_edge(xge, L0, We, 256, 256, KO, KO, 256)
    OUT = _k_final(MXe, MNe, Se, ge, be, L0)

    OUT = OUT.reshape(NI, 1024)
    xo = OUT[0:2][:, :, None]
    yo = OUT[2:4][:, :, None]
    return xo, yo
